# baseline (speedup 1.0000x reference)
import functools

import jax
import jax.numpy as jnp
from jax import lax
from jax.experimental import pallas as pl
from jax.experimental.pallas import tpu as pltpu
from jax.experimental.pallas import tpu_sc as plsc

N_PRED = 10000
N_ITEM = 10000
N = 20000
EMB = 128
E = 320000

NC = 2
NS = 16
L = 16

NPAD = 20480
ZSTRIPE = NPAD // NS
HALF = N // NC
ACC_ROWS = 10112
ACC_STRIPE = ACC_ROWS // NS

C = 400
GRP = C // L
CW = 160
GRPW = CW // L
PT_ALL = E // (NC * NS)
PT_SC = E // NS

_mesh = plsc.VectorSubcoreMesh(core_axis_name="c", subcore_axis_name="s")
_f32 = jnp.float32
_i32 = jnp.int32


_GDN = lax.GatherDimensionNumbers(offset_dims=(), collapsed_slice_dims=(0,),
                                  start_index_map=(0,))


def _perm16(x, idx):
    return lax.gather(x, idx[:, None], _GDN, slice_sizes=(1,),
                      mode=lax.GatherScatterMode.PROMISE_IN_BOUNDS)


def _hsum_all(x):
    lane = lax.iota(_i32, L)
    for step in (8, 4, 2, 1):
        x = x + _perm16(x, lane ^ step)
    return x


def _zero_vec(ref, n):
    def body(i, _):
        ref[pl.ds(i * L, L)] = jnp.zeros((L,), _f32)
        return 0
    lax.fori_loop(0, n // L, body, 0)


def _zero_rows(ref, rows):
    def body(i, _):
        for j in range(EMB // L):
            ref[i, pl.ds(j * L, L)] = jnp.zeros((L,), _f32)
        return 0
    lax.fori_loop(0, rows, body, 0)


def _gat_scores_body(el_h, er_h, a_h, src_h, dst_h, ee_h, zp_h,
                     idxs_v, idxd_v, ee_v, Lb, Rb, a_v, z_sh, sem1, sem2):
    c = lax.axis_index("c")
    s = lax.axis_index("s")
    wid = s * NC + c

    _zero_vec(ee_v, C)
    for off in (0, 400, 800):
        pltpu.sync_copy(ee_v, z_sh.at[pl.ds(s * ZSTRIPE + off, 400)])
    pltpu.sync_copy(ee_v.at[pl.ds(0, 80)], z_sh.at[pl.ds(s * ZSTRIPE + 1200, 80)])
    plsc.subcore_barrier()

    pltpu.sync_copy(a_h, a_v)

    def chunk(i, _):
        base = wid * PT_ALL + i * C
        pltpu.sync_copy(src_h.at[pl.ds(base, C)], idxs_v)
        pltpu.sync_copy(dst_h.at[pl.ds(base, C)], idxd_v)
        cp1 = pltpu.async_copy(el_h.at[idxs_v], Lb, sem1)
        cp2 = pltpu.async_copy(er_h.at[idxd_v], Rb, sem2)
        cp1.wait()
        cp2.wait()

        def group(g, _):
            a_vecs = [a_v[pl.ds(j * L, L)] for j in range(EMB // L)]
            lane = jax.lax.iota(_i32, L)
            sc16 = jnp.zeros((L,), _f32)
            for k in range(L):
                row = g * L + k
                acc_l = jnp.zeros((L,), _f32)
                acc_a = jnp.zeros((L,), _f32)
                for j in range(EMB // L):
                    sv = Lb[row, pl.ds(j * L, L)] + Rb[row, pl.ds(j * L, L)]
                    acc_l = acc_l + a_vecs[j] * sv
                    acc_a = acc_a + a_vecs[j] * jnp.abs(sv)
                sk = _hsum_all(0.6 * acc_l + 0.4 * acc_a)
                sc16 = jnp.where(lane == k, sk, sc16)
            ee_v[pl.ds(g * L, L)] = jnp.exp(sc16)
            return 0

        lax.fori_loop(0, GRP, group, 0)
        pltpu.sync_copy(ee_v, ee_h.at[pl.ds(base, C)])
        pltpu.sync_copy(ee_v, z_sh.at[idxd_v], add=True)
        return 0

    lax.fori_loop(0, PT_ALL // C, chunk, 0)
    plsc.subcore_barrier()
    pltpu.sync_copy(z_sh.at[pl.ds(s * ZSTRIPE, ZSTRIPE)],
                    zp_h.at[pl.ds(c * NPAD + s * ZSTRIPE, ZSTRIPE)])


def _sc_gat_scores(el, er, a, src, dst):
    fn = pl.kernel(
        _gat_scores_body,
        out_type=(jax.ShapeDtypeStruct((E,), _f32),
                  jax.ShapeDtypeStruct((NC * NPAD,), _f32)),
        scratch_types=[
            pltpu.VMEM((C,), _i32),
            pltpu.VMEM((C,), _i32),
            pltpu.VMEM((C,), _f32),
            pltpu.VMEM((C, EMB), _f32),
            pltpu.VMEM((C, EMB), _f32),
            pltpu.VMEM((EMB,), _f32),
            pltpu.VMEM_SHARED((NPAD,), _f32),
            pltpu.SemaphoreType.DMA,
            pltpu.SemaphoreType.DMA,
        ],
        mesh=_mesh,
    )
    return fn(el, er, a, src, dst)


def _deg_body(dst_h, zp_h, idxd_v, ones_v, z_sh, sem1):
    c = lax.axis_index("c")
    s = lax.axis_index("s")
    wid = s * NC + c

    _zero_vec(ones_v, C)
    for off in (0, 400, 800):
        pltpu.sync_copy(ones_v, z_sh.at[pl.ds(s * ZSTRIPE + off, 400)])
    pltpu.sync_copy(ones_v.at[pl.ds(0, 80)], z_sh.at[pl.ds(s * ZSTRIPE + 1200, 80)])
    plsc.subcore_barrier()

    def fill(i, _):
        ones_v[pl.ds(i * L, L)] = jnp.ones((L,), _f32)
        return 0
    lax.fori_loop(0, C // L, fill, 0)

    def chunk(i, _):
        base = wid * PT_ALL + i * C
        pltpu.sync_copy(dst_h.at[pl.ds(base, C)], idxd_v)
        pltpu.sync_copy(ones_v, z_sh.at[idxd_v], add=True)
        return 0

    lax.fori_loop(0, PT_ALL // C, chunk, 0)
    plsc.subcore_barrier()
    pltpu.sync_copy(z_sh.at[pl.ds(s * ZSTRIPE, ZSTRIPE)],
                    zp_h.at[pl.ds(c * NPAD + s * ZSTRIPE, ZSTRIPE)])


def _sc_deg(dst):
    fn = pl.kernel(
        _deg_body,
        out_type=jax.ShapeDtypeStruct((NC * NPAD,), _f32),
        scratch_types=[
            pltpu.VMEM((C,), _i32),
            pltpu.VMEM((C,), _f32),
            pltpu.VMEM_SHARED((NPAD,), _f32),
            pltpu.SemaphoreType.DMA,
        ],
        mesh=_mesh,
    )
    return fn(dst)


def _wscatter_body(weighted, rows_h, w_h, zinv_h, src_h, dst_h, out_h,
                   idxs_v, idxd_v, idxloc_v, w_v, zw_v, Vb, acc_sh, sem1, sem2):
    c = lax.axis_index("c")
    s = lax.axis_index("s")
    half0 = c * HALF

    _zero_rows(Vb, CW)
    for off in (0, 160, 320, 480):
        ln = min(CW, ACC_STRIPE - off)
        pltpu.sync_copy(Vb.at[pl.ds(0, ln)],
                        acc_sh.at[pl.ds(s * ACC_STRIPE + off, ln)])
    plsc.subcore_barrier()

    def chunk(i, _):
        base = s * PT_SC + i * CW
        pltpu.sync_copy(src_h.at[pl.ds(base, CW)], idxs_v)
        pltpu.sync_copy(dst_h.at[pl.ds(base, CW)], idxd_v)
        if weighted:
            pltpu.sync_copy(w_h.at[pl.ds(base, CW)], w_v)
        cp1 = pltpu.async_copy(rows_h.at[idxs_v], Vb, sem1)
        if weighted:
            pltpu.async_copy(zinv_h.at[idxd_v], zw_v, sem2).wait()
        cp1.wait()

        def group(g, _):
            d16 = idxd_v[pl.ds(g * L, L)]
            inhalf = (d16 >= half0) & (d16 < half0 + HALF)
            loc = jnp.where(inhalf, d16 - half0, HALF)
            idxloc_v[pl.ds(g * L, L)] = loc
            if weighted:
                wv = w_v[pl.ds(g * L, L)] * zw_v[pl.ds(g * L, L)]
                for k in range(L):
                    row = g * L + k
                    wk = wv[k]
                    for j in range(EMB // L):
                        Vb[row, pl.ds(j * L, L)] = Vb[row, pl.ds(j * L, L)] * wk
            return 0

        lax.fori_loop(0, GRPW, group, 0)
        pltpu.sync_copy(Vb, acc_sh.at[idxloc_v], add=True)
        return 0

    lax.fori_loop(0, PT_SC // CW, chunk, 0)
    plsc.subcore_barrier()

    @pl.when(s < NS - 1)
    def _():
        pltpu.sync_copy(acc_sh.at[pl.ds(s * ACC_STRIPE, ACC_STRIPE)],
                        out_h.at[pl.ds(c * HALF + s * ACC_STRIPE, ACC_STRIPE)])

    @pl.when(s == NS - 1)
    def _():
        pltpu.sync_copy(acc_sh.at[pl.ds((NS - 1) * ACC_STRIPE, HALF - (NS - 1) * ACC_STRIPE)],
                        out_h.at[pl.ds(c * HALF + (NS - 1) * ACC_STRIPE, HALF - (NS - 1) * ACC_STRIPE)])


def _sc_wscatter(rows, w, zinv, src, dst, weighted):
    fn = pl.kernel(
        functools.partial(_wscatter_body, weighted),
        out_type=jax.ShapeDtypeStruct((N, EMB), _f32),
        scratch_types=[
            pltpu.VMEM((CW,), _i32),
            pltpu.VMEM((CW,), _i32),
            pltpu.VMEM((CW,), _i32),
            pltpu.VMEM((CW,), _f32),
            pltpu.VMEM((CW,), _f32),
            pltpu.VMEM((CW, EMB), _f32),
            pltpu.VMEM_SHARED((ACC_ROWS, EMB), _f32),
            pltpu.SemaphoreType.DMA,
            pltpu.SemaphoreType.DMA,
        ],
        mesh=_mesh,
    )
    return fn(rows, w, zinv, src, dst)


def _edge_dot_body(a_h, b_h, src_h, dst_h, out_h,
                   idxs_v, idxd_v, out_v, Ab, Bb, sem1, sem2):
    c = lax.axis_index("c")
    s = lax.axis_index("s")
    wid = s * NC + c

    def chunk(i, _):
        base = wid * PT_ALL + i * C
        pltpu.sync_copy(src_h.at[pl.ds(base, C)], idxs_v)
        pltpu.sync_copy(dst_h.at[pl.ds(base, C)], idxd_v)
        cp1 = pltpu.async_copy(a_h.at[idxs_v], Ab, sem1)
        cp2 = pltpu.async_copy(b_h.at[idxd_v], Bb, sem2)
        cp1.wait()
        cp2.wait()

        def group(g, _):
            lane = jax.lax.iota(_i32, L)
            dot16 = jnp.zeros((L,), _f32)
            for k in range(L):
                row = g * L + k
                acc = jnp.zeros((L,), _f32)
                for j in range(EMB // L):
                    acc = acc + Ab[row, pl.ds(j * L, L)] * Bb[row, pl.ds(j * L, L)]
                dot16 = jnp.where(lane == k, _hsum_all(acc), dot16)
            out_v[pl.ds(g * L, L)] = dot16
            return 0

        lax.fori_loop(0, GRP, group, 0)
        pltpu.sync_copy(out_v, out_h.at[pl.ds(base, C)])
        return 0

    lax.fori_loop(0, PT_ALL // C, chunk, 0)


def _sc_edge_dot(a, b, src, dst):
    fn = pl.kernel(
        _edge_dot_body,
        out_type=jax.ShapeDtypeStruct((E,), _f32),
        scratch_types=[
            pltpu.VMEM((C,), _i32),
            pltpu.VMEM((C,), _i32),
            pltpu.VMEM((C,), _f32),
            pltpu.VMEM((C, EMB), _f32),
            pltpu.VMEM((C, EMB), _f32),
            pltpu.SemaphoreType.DMA,
            pltpu.SemaphoreType.DMA,
        ],
        mesh=_mesh,
    )
    return fn(a, b, src, dst)


BM = 2000
GRID = N // BM


def _lrelu(x, s):
    return jnp.where(x >= 0, x, s * x)


def _row_spec():
    return pl.BlockSpec((BM, EMB), lambda i: (i, 0))


def _const_spec(shape):
    return pl.BlockSpec(shape, lambda i: (0, 0))


def _apply_act(x, act):
    if act == 'lrelu':
        return _lrelu(x, 0.01)
    return x


def _tc_linear(inputs, weights, biases, out_lrelu=False):
    nin, nout = len(inputs), len(weights)
    acts = tuple(a for _, a in inputs)

    def body(*refs):
        in_refs = refs[:nin]
        w_refs = refs[nin:nin + nin * nout]
        b_refs = refs[nin + nin * nout:nin + nin * nout + nout]
        out_refs = refs[nin + nin * nout + nout:]
        xs = [_apply_act(r[...], a) for r, a in zip(in_refs, acts)]
        for o in range(nout):
            acc = jnp.broadcast_to(b_refs[o][...], (BM, EMB))
            for i in range(nin):
                acc = acc + jnp.dot(xs[i], w_refs[o * nin + i][...],
                                    preferred_element_type=_f32)
            out_refs[o][...] = _lrelu(acc, 0.01) if out_lrelu else acc

    args = ([x for x, _ in inputs]
            + [w for per_out in weights for w in per_out]
            + [b.reshape(1, EMB) for b in biases])
    in_specs = ([_row_spec() for _ in range(nin)]
                + [_const_spec((EMB, EMB)) for _ in range(nin * nout)]
                + [_const_spec((1, EMB)) for _ in range(nout)])
    out = pl.pallas_call(
        body,
        grid=(GRID,),
        in_specs=in_specs,
        out_specs=[_row_spec() for _ in range(nout)],
        out_shape=[jax.ShapeDtypeStruct((N, EMB), _f32) for _ in range(nout)],
    )(*args)
    return out if nout > 1 else out[0]


def _tc_stats_body(x_ref, o_ref):
    @pl.when(pl.program_id(0) == 0)
    def _():
        o_ref[...] = jnp.zeros((8, EMB), _f32)
    x = x_ref[...]
    o_ref[0, :] += jnp.sum(x, axis=0)
    o_ref[1, :] += jnp.sum(x * x, axis=0)


def _tc_stats(x):
    return pl.pallas_call(
        _tc_stats_body,
        grid=(GRID,),
        in_specs=[_row_spec()],
        out_specs=pl.BlockSpec((8, EMB), lambda i: (0, 0)),
        out_shape=jax.ShapeDtypeStruct((8, EMB), _f32),
    )(x)


def _bn_from_stats(x, s_ref, g_ref, be_ref):
    mu = s_ref[0, :] * (1.0 / N)
    var = s_ref[1, :] * (1.0 / N) - mu * mu
    rstd = jax.lax.rsqrt(var + 1e-5)
    return (x - mu[None, :]) * (rstd * g_ref[0, :])[None, :] + be_ref[0, :][None, :]


def _tc_bn(x, stats, g, be, out_lrelu):
    def body(x_ref, s_ref, g_ref, b_ref, o_ref):
        y = _bn_from_stats(x_ref[...], s_ref, g_ref, b_ref)
        o_ref[...] = _lrelu(y, 0.01) if out_lrelu else y

    return pl.pallas_call(
        body,
        grid=(GRID,),
        in_specs=[_row_spec(), _const_spec((8, EMB)),
                  _const_spec((1, EMB)), _const_spec((1, EMB))],
        out_specs=_row_spec(),
        out_shape=jax.ShapeDtypeStruct((N, EMB), _f32),
    )(x, stats, g.reshape(1, EMB), be.reshape(1, EMB))


def _tc_bn_add(x1, s1, g1, be1, x2, s2, g2, be2):
    def body(x1r, s1r, g1r, b1r, x2r, s2r, g2r, b2r, o_ref):
        y1 = _lrelu(_bn_from_stats(x1r[...], s1r, g1r, b1r), 0.01)
        y2 = _lrelu(_bn_from_stats(x2r[...], s2r, g2r, b2r), 0.01)
        o_ref[...] = y1 + y2

    return pl.pallas_call(
        body,
        grid=(GRID,),
        in_specs=[_row_spec(), _const_spec((8, EMB)), _const_spec((1, EMB)),
                  _const_spec((1, EMB))] * 2,
        out_specs=_row_spec(),
        out_shape=jax.ShapeDtypeStruct((N, EMB), _f32),
    )(x1, s1, g1.reshape(1, EMB), be1.reshape(1, EMB),
      x2, s2, g2.reshape(1, EMB), be2.reshape(1, EMB))


def _tc_soc_emb(i2u_raw, ui):
    def body(a_ref, u_ref, o_ref):
        t = _lrelu(a_ref[...], 0.01)
        m = jnp.sum(t, axis=1, keepdims=True) != 0
        o_ref[...] = jnp.where(m, t, u_ref[...])

    return pl.pallas_call(
        body,
        grid=(GRID,),
        in_specs=[_row_spec(), _row_spec()],
        out_specs=_row_spec(),
        out_shape=jax.ShapeDtypeStruct((N, EMB), _f32),
    )(i2u_raw, ui)


def _tc_cheb_step1(x0, agg0, dinv2d, c1):
    def body(x0r, ar, dr, c1r, x1r, hdr):
        c = c1r[0, 0]
        d = dr[...]
        x0 = x0r[...]
        x1 = c * (x0 - ar[...] * d) - x0
        x1r[...] = x1
        hdr[...] = x1 * d

    return pl.pallas_call(
        body,
        grid=(GRID,),
        in_specs=[_row_spec(), _row_spec(), pl.BlockSpec((BM, 1), lambda i: (i, 0)),
                  pl.BlockSpec(memory_space=pltpu.SMEM)],
        out_specs=[_row_spec(), _row_spec()],
        out_shape=[jax.ShapeDtypeStruct((N, EMB), _f32)] * 2,
    )(x0, agg0, dinv2d, c1)


def _tc_cheb_step2(x1, agg1, x0, dinv2d, c1):
    def body(x1r, ar, x0r, dr, c1r, o_ref):
        c = c1r[0, 0]
        x1 = x1r[...]
        o_ref[...] = 2.0 * (c * (x1 - ar[...] * dr[...]) - x1) - x0r[...]

    return pl.pallas_call(
        body,
        grid=(GRID,),
        in_specs=[_row_spec(), _row_spec(), _row_spec(),
                  pl.BlockSpec((BM, 1), lambda i: (i, 0)),
                  pl.BlockSpec(memory_space=pltpu.SMEM)],
        out_specs=_row_spec(),
        out_shape=jax.ShapeDtypeStruct((N, EMB), _f32),
    )(x1, agg1, x0, dinv2d, c1)


def _tc_colscale(x, dinv2d):
    def body(xr, dr, o_ref):
        o_ref[...] = xr[...] * dr[...]

    return pl.pallas_call(
        body,
        grid=(GRID,),
        in_specs=[_row_spec(), pl.BlockSpec((BM, 1), lambda i: (i, 0))],
        out_specs=_row_spec(),
        out_shape=jax.ShapeDtypeStruct((N, EMB), _f32),
    )(x, dinv2d)


def _tc_mutmix(h_uP, h_uS):
    def body(pr, sr, apr, asr):
        p = pr[...]
        s = sr[...]
        hm = p * s

        def sm(x):
            ex = jnp.exp(x - jnp.max(x, axis=1, keepdims=True))
            return ex / jnp.sum(ex, axis=1, keepdims=True)

        apr[...] = hm * sm(p)
        asr[...] = hm * sm(s)

    return pl.pallas_call(
        body,
        grid=(GRID,),
        in_specs=[_row_spec(), _row_spec()],
        out_specs=[_row_spec(), _row_spec()],
        out_shape=[jax.ShapeDtypeStruct((N, EMB), _f32)] * 2,
    )(h_uP, h_uS)


def _mlp_bn_T(x_acts, p):
    k = len(x_acts)
    Ws = [p['W'][i * EMB:(i + 1) * EMB] for i in range(k)]
    T = _tc_linear(x_acts, [Ws], [p['b']])
    return T, _tc_stats(T)


def _gatv2_raw(x_act, src, dst, p):
    el, er = _tc_linear([x_act], [[p['Ws']], [p['Wd']]], [p['bs'], p['bd']])
    ee, zpart = _sc_gat_scores(el, er, p['a'], src, dst)
    z = zpart[:N] + zpart[NPAD:NPAD + N]
    zinv = 1.0 / (z + 1e-9)
    return _sc_wscatter(el, ee, zinv, src, dst, weighted=True)


def _cheb(x, src, dst, W, b, c1, dinv2d):
    zw = jnp.zeros((E,), _f32)
    zn = jnp.zeros((N,), _f32)
    X0 = x
    hd0 = _tc_colscale(X0, dinv2d)
    agg0 = _sc_wscatter(hd0, zw, zn, src, dst, weighted=False)
    X1, hd1 = _tc_cheb_step1(X0, agg0, dinv2d, c1)
    agg1 = _sc_wscatter(hd1, zw, zn, src, dst, weighted=False)
    X2 = _tc_cheb_step2(X1, agg1, X0, dinv2d, c1)
    return _tc_linear([(X0, None), (X1, None), (X2, None)],
                      [[W[0], W[1], W[2]]], [b], out_lrelu=True)


def kernel(params, laplacian_lambda_max, g_edge_index, user2item_edge_index,
           reverse_edge_index, item2user_edge_index, social_edge_index):
    lam = laplacian_lambda_max[0]
    c1 = (2.0 / lam).reshape(1, 1)

    S0 = _tc_stats(params['emb'])
    ui = _tc_bn(params['emb'], S0, params['bn_g'], params['bn_b'], out_lrelu=False)

    u2i = user2item_edge_index
    rev = reverse_edge_index
    i2u_ei = item2user_edge_index
    soc = social_edge_index

    h_raw = _gatv2_raw((ui, None), u2i[0], u2i[1], params['gat_u2i'])
    ii_raw = _gatv2_raw((h_raw, 'lrelu'), rev[0], rev[1], params['gat_ii'])
    i2u_raw = _gatv2_raw((ui, None), i2u_ei[0], i2u_ei[1], params['gat_i2u'])
    soc_emb = _tc_soc_emb(i2u_raw, ui)
    si_raw = _gatv2_raw((soc_emb, None), soc[0], soc[1], params['gat_si'])

    T_sp, S_sp = _mlp_bn_T([(ii_raw, 'lrelu'), (si_raw, 'lrelu')],
                           params['spatial_out'])
    spatial = _tc_bn(T_sp, S_sp, params['spatial_out']['g'],
                     params['spatial_out']['be'], out_lrelu=True)

    degpart = _sc_deg(soc[1])
    deg = degpart[:N] + degpart[NPAD:NPAD + N]
    dinv2d = jnp.where(deg > 0, 1.0 / jnp.sqrt(jnp.maximum(deg, 1e-9)),
                       0.0).reshape(N, 1)
    hs = _cheb(ui, soc[0], soc[1], params['cheb_W'], params['cheb_b'], c1, dinv2d)
    hs = _cheb(hs, soc[0], soc[1], params['cheb_W'], params['cheb_b'], c1, dinv2d)
    sp_raw = _gatv2_raw((hs, None), soc[0], soc[1], params['gat_spec'])

    T_P, S_P = _mlp_bn_T([(spatial, None), (ui, None)], params['mut_c'])
    h_uP = _tc_bn(T_P, S_P, params['mut_c']['g'], params['mut_c']['be'],
                  out_lrelu=True)
    T_S, S_S = _mlp_bn_T([(sp_raw, 'lrelu'), (ui, None)], params['mut_s'])
    h_uS = _tc_bn(T_S, S_S, params['mut_s']['g'], params['mut_s']['be'],
                  out_lrelu=True)

    A_P, A_S = _tc_mutmix(h_uP, h_uS)
    T_nP, S_nP = _mlp_bn_T([(A_P, None), (h_uP, None)], params['pred_p'])
    T_nS, S_nS = _mlp_bn_T([(A_S, None), (h_uS, None)], params['pred_s'])
    new_ft = _tc_bn_add(T_nP, S_nP, params['pred_p']['g'], params['pred_p']['be'],
                        T_nS, S_nS, params['pred_s']['g'], params['pred_s']['be'])

    T_r, S_r = _mlp_bn_T([(ui, None)], params['raw'])
    raw_ft = _tc_bn(T_r, S_r, params['raw']['g'], params['raw']['be'],
                    out_lrelu=True)

    ed = _sc_edge_dot(new_ft, raw_ft, g_edge_index[0], g_edge_index[1])
    return ed.reshape(E, 1)


# speedup vs baseline: 1.0185x; 1.0185x over previous
import functools

import jax
import jax.numpy as jnp
from jax import lax
from jax.experimental import pallas as pl
from jax.experimental.pallas import tpu as pltpu
from jax.experimental.pallas import tpu_sc as plsc

N_PRED = 10000
N_ITEM = 10000
N = 20000
EMB = 128
E = 320000

NC = 2
NS = 16
L = 16

NPAD = 20480
ZSTRIPE = NPAD // NS
HALF = N // NC
ACC_ROWS = 10112
ACC_STRIPE = ACC_ROWS // NS

C = 400
GRP = C // L
CS = 80
GRPS = CS // L
CW = 160
GRPW = CW // L
PT_ALL = E // (NC * NS)
PT_SC = E // NS

_mesh = plsc.VectorSubcoreMesh(core_axis_name="c", subcore_axis_name="s")
_f32 = jnp.float32
_i32 = jnp.int32


_GDN = lax.GatherDimensionNumbers(offset_dims=(), collapsed_slice_dims=(0,),
                                  start_index_map=(0,))


def _perm16(x, idx):
    return lax.gather(x, idx[:, None], _GDN, slice_sizes=(1,),
                      mode=lax.GatherScatterMode.PROMISE_IN_BOUNDS)


def _hsum_all(x):
    lane = lax.iota(_i32, L)
    for step in (8, 4, 2, 1):
        x = x + _perm16(x, lane ^ step)
    return x


def _zero_vec(ref, n):
    def body(i, _):
        ref[pl.ds(i * L, L)] = jnp.zeros((L,), _f32)
        return 0
    lax.fori_loop(0, n // L, body, 0)


def _zero_rows(ref, rows):
    def body(i, _):
        for j in range(EMB // L):
            ref[i, pl.ds(j * L, L)] = jnp.zeros((L,), _f32)
        return 0
    lax.fori_loop(0, rows, body, 0)


def _gat_scores_body(el_h, er_h, a_h, src_h, dst_h, ee_h, zp_h,
                     idxs0, idxd0, idxs1, idxd1, ee_v0, ee_v1,
                     Lb0, Rb0, Lb1, Rb1, a_v, z_sh,
                     semL0, semR0, semL1, semR1):
    c = lax.axis_index("c")
    s = lax.axis_index("s")
    wid = s * NC + c

    _zero_vec(ee_v0, CS)
    for off in range(0, ZSTRIPE, CS):
        pltpu.sync_copy(ee_v0, z_sh.at[pl.ds(s * ZSTRIPE + off, CS)])
    plsc.subcore_barrier()

    pltpu.sync_copy(a_h, a_v)

    sets = ((idxs0, idxd0, ee_v0, Lb0, Rb0, semL0, semR0),
            (idxs1, idxd1, ee_v1, Lb1, Rb1, semL1, semR1))

    def load_and_fire(t, i):
        idxs_v, idxd_v, _, Lb, Rb, sL, sR = sets[t]
        base = wid * PT_ALL + i * CS
        pltpu.sync_copy(src_h.at[pl.ds(base, CS)], idxs_v)
        pltpu.sync_copy(dst_h.at[pl.ds(base, CS)], idxd_v)
        pltpu.async_copy(el_h.at[idxs_v], Lb, sL)
        pltpu.async_copy(er_h.at[idxd_v], Rb, sR)

    def compute(t, i):
        idxs_v, idxd_v, ee_v, Lb, Rb, sL, sR = sets[t]
        base = wid * PT_ALL + i * CS
        pltpu.make_async_copy(el_h.at[idxs_v], Lb, sL).wait()
        pltpu.make_async_copy(er_h.at[idxd_v], Rb, sR).wait()

        def group(g, _):
            a_vecs = [a_v[pl.ds(j * L, L)] for j in range(EMB // L)]
            lane = jax.lax.iota(_i32, L)
            sc16 = jnp.zeros((L,), _f32)
            for k in range(L):
                row = g * L + k
                acc_l = jnp.zeros((L,), _f32)
                acc_a = jnp.zeros((L,), _f32)
                for j in range(EMB // L):
                    sv = Lb[row, pl.ds(j * L, L)] + Rb[row, pl.ds(j * L, L)]
                    acc_l = acc_l + a_vecs[j] * sv
                    acc_a = acc_a + a_vecs[j] * jnp.abs(sv)
                sk = _hsum_all(0.6 * acc_l + 0.4 * acc_a)
                sc16 = jnp.where(lane == k, sk, sc16)
            ee_v[pl.ds(g * L, L)] = jnp.exp(sc16)
            return 0

        lax.fori_loop(0, GRPS, group, 0)
        pltpu.sync_copy(ee_v, ee_h.at[pl.ds(base, CS)])
        pltpu.sync_copy(ee_v, z_sh.at[idxd_v], add=True)

    nch = PT_ALL // CS

    load_and_fire(0, 0)

    def pair(p, _):
        load_and_fire(1, 2 * p + 1)
        compute(0, 2 * p)
        load_and_fire(0, 2 * p + 2)
        compute(1, 2 * p + 1)
        return 0

    lax.fori_loop(0, nch // 2, pair, 0)
    compute(0, nch - 1)

    plsc.subcore_barrier()
    pltpu.sync_copy(z_sh.at[pl.ds(s * ZSTRIPE, ZSTRIPE)],
                    zp_h.at[pl.ds(c * NPAD + s * ZSTRIPE, ZSTRIPE)])


def _sc_gat_scores(el, er, a, src, dst):
    fn = pl.kernel(
        _gat_scores_body,
        out_type=(jax.ShapeDtypeStruct((E,), _f32),
                  jax.ShapeDtypeStruct((NC * NPAD,), _f32)),
        scratch_types=[
            pltpu.VMEM((CS,), _i32),
            pltpu.VMEM((CS,), _i32),
            pltpu.VMEM((CS,), _i32),
            pltpu.VMEM((CS,), _i32),
            pltpu.VMEM((CS,), _f32),
            pltpu.VMEM((CS,), _f32),
            pltpu.VMEM((CS, EMB), _f32),
            pltpu.VMEM((CS, EMB), _f32),
            pltpu.VMEM((CS, EMB), _f32),
            pltpu.VMEM((CS, EMB), _f32),
            pltpu.VMEM((EMB,), _f32),
            pltpu.VMEM_SHARED((NPAD,), _f32),
            pltpu.SemaphoreType.DMA,
            pltpu.SemaphoreType.DMA,
            pltpu.SemaphoreType.DMA,
            pltpu.SemaphoreType.DMA,
        ],
        mesh=_mesh,
    )
    return fn(el, er, a, src, dst)


def _deg_body(dst_h, zp_h, idxd_v, ones_v, z_sh, sem1):
    c = lax.axis_index("c")
    s = lax.axis_index("s")
    wid = s * NC + c

    _zero_vec(ones_v, C)
    for off in (0, 400, 800):
        pltpu.sync_copy(ones_v, z_sh.at[pl.ds(s * ZSTRIPE + off, 400)])
    pltpu.sync_copy(ones_v.at[pl.ds(0, 80)], z_sh.at[pl.ds(s * ZSTRIPE + 1200, 80)])
    plsc.subcore_barrier()

    def fill(i, _):
        ones_v[pl.ds(i * L, L)] = jnp.ones((L,), _f32)
        return 0
    lax.fori_loop(0, C // L, fill, 0)

    def chunk(i, _):
        base = wid * PT_ALL + i * C
        pltpu.sync_copy(dst_h.at[pl.ds(base, C)], idxd_v)
        pltpu.sync_copy(ones_v, z_sh.at[idxd_v], add=True)
        return 0

    lax.fori_loop(0, PT_ALL // C, chunk, 0)
    plsc.subcore_barrier()
    pltpu.sync_copy(z_sh.at[pl.ds(s * ZSTRIPE, ZSTRIPE)],
                    zp_h.at[pl.ds(c * NPAD + s * ZSTRIPE, ZSTRIPE)])


def _sc_deg(dst):
    fn = pl.kernel(
        _deg_body,
        out_type=jax.ShapeDtypeStruct((NC * NPAD,), _f32),
        scratch_types=[
            pltpu.VMEM((C,), _i32),
            pltpu.VMEM((C,), _f32),
            pltpu.VMEM_SHARED((NPAD,), _f32),
            pltpu.SemaphoreType.DMA,
        ],
        mesh=_mesh,
    )
    return fn(dst)


def _wscatter_body(weighted, rows_h, w_h, zinv_h, src_h, dst_h, out_h,
                   idxs_v, idxd_v, idxloc_v, w_v, zw_v, Vb, acc_sh, sem1, sem2):
    c = lax.axis_index("c")
    s = lax.axis_index("s")
    half0 = c * HALF

    _zero_rows(Vb, CW)
    for off in (0, 160, 320, 480):
        ln = min(CW, ACC_STRIPE - off)
        pltpu.sync_copy(Vb.at[pl.ds(0, ln)],
                        acc_sh.at[pl.ds(s * ACC_STRIPE + off, ln)])
    plsc.subcore_barrier()

    def chunk(i, _):
        base = s * PT_SC + i * CW
        pltpu.sync_copy(src_h.at[pl.ds(base, CW)], idxs_v)
        pltpu.sync_copy(dst_h.at[pl.ds(base, CW)], idxd_v)
        if weighted:
            pltpu.sync_copy(w_h.at[pl.ds(base, CW)], w_v)
        cp1 = pltpu.async_copy(rows_h.at[idxs_v], Vb, sem1)
        if weighted:
            pltpu.async_copy(zinv_h.at[idxd_v], zw_v, sem2).wait()
        cp1.wait()

        def group(g, _):
            d16 = idxd_v[pl.ds(g * L, L)]
            inhalf = (d16 >= half0) & (d16 < half0 + HALF)
            loc = jnp.where(inhalf, d16 - half0, HALF)
            idxloc_v[pl.ds(g * L, L)] = loc
            if weighted:
                wv = w_v[pl.ds(g * L, L)] * zw_v[pl.ds(g * L, L)]
                for k in range(L):
                    row = g * L + k
                    wk = wv[k]
                    for j in range(EMB // L):
                        Vb[row, pl.ds(j * L, L)] = Vb[row, pl.ds(j * L, L)] * wk
            return 0

        lax.fori_loop(0, GRPW, group, 0)
        pltpu.sync_copy(Vb, acc_sh.at[idxloc_v], add=True)
        return 0

    lax.fori_loop(0, PT_SC // CW, chunk, 0)
    plsc.subcore_barrier()

    @pl.when(s < NS - 1)
    def _():
        pltpu.sync_copy(acc_sh.at[pl.ds(s * ACC_STRIPE, ACC_STRIPE)],
                        out_h.at[pl.ds(c * HALF + s * ACC_STRIPE, ACC_STRIPE)])

    @pl.when(s == NS - 1)
    def _():
        pltpu.sync_copy(acc_sh.at[pl.ds((NS - 1) * ACC_STRIPE, HALF - (NS - 1) * ACC_STRIPE)],
                        out_h.at[pl.ds(c * HALF + (NS - 1) * ACC_STRIPE, HALF - (NS - 1) * ACC_STRIPE)])


def _sc_wscatter(rows, w, zinv, src, dst, weighted):
    fn = pl.kernel(
        functools.partial(_wscatter_body, weighted),
        out_type=jax.ShapeDtypeStruct((N, EMB), _f32),
        scratch_types=[
            pltpu.VMEM((CW,), _i32),
            pltpu.VMEM((CW,), _i32),
            pltpu.VMEM((CW,), _i32),
            pltpu.VMEM((CW,), _f32),
            pltpu.VMEM((CW,), _f32),
            pltpu.VMEM((CW, EMB), _f32),
            pltpu.VMEM_SHARED((ACC_ROWS, EMB), _f32),
            pltpu.SemaphoreType.DMA,
            pltpu.SemaphoreType.DMA,
        ],
        mesh=_mesh,
    )
    return fn(rows, w, zinv, src, dst)


def _edge_dot_body(a_h, b_h, src_h, dst_h, out_h,
                   idxs_v, idxd_v, out_v, Ab, Bb, sem1, sem2):
    c = lax.axis_index("c")
    s = lax.axis_index("s")
    wid = s * NC + c

    def chunk(i, _):
        base = wid * PT_ALL + i * C
        pltpu.sync_copy(src_h.at[pl.ds(base, C)], idxs_v)
        pltpu.sync_copy(dst_h.at[pl.ds(base, C)], idxd_v)
        cp1 = pltpu.async_copy(a_h.at[idxs_v], Ab, sem1)
        cp2 = pltpu.async_copy(b_h.at[idxd_v], Bb, sem2)
        cp1.wait()
        cp2.wait()

        def group(g, _):
            lane = jax.lax.iota(_i32, L)
            dot16 = jnp.zeros((L,), _f32)
            for k in range(L):
                row = g * L + k
                acc = jnp.zeros((L,), _f32)
                for j in range(EMB // L):
                    acc = acc + Ab[row, pl.ds(j * L, L)] * Bb[row, pl.ds(j * L, L)]
                dot16 = jnp.where(lane == k, _hsum_all(acc), dot16)
            out_v[pl.ds(g * L, L)] = dot16
            return 0

        lax.fori_loop(0, GRP, group, 0)
        pltpu.sync_copy(out_v, out_h.at[pl.ds(base, C)])
        return 0

    lax.fori_loop(0, PT_ALL // C, chunk, 0)


def _sc_edge_dot(a, b, src, dst):
    fn = pl.kernel(
        _edge_dot_body,
        out_type=jax.ShapeDtypeStruct((E,), _f32),
        scratch_types=[
            pltpu.VMEM((C,), _i32),
            pltpu.VMEM((C,), _i32),
            pltpu.VMEM((C,), _f32),
            pltpu.VMEM((C, EMB), _f32),
            pltpu.VMEM((C, EMB), _f32),
            pltpu.SemaphoreType.DMA,
            pltpu.SemaphoreType.DMA,
        ],
        mesh=_mesh,
    )
    return fn(a, b, src, dst)


BM = 2000
GRID = N // BM


def _lrelu(x, s):
    return jnp.where(x >= 0, x, s * x)


def _row_spec():
    return pl.BlockSpec((BM, EMB), lambda i: (i, 0))


def _const_spec(shape):
    return pl.BlockSpec(shape, lambda i: (0, 0))


def _apply_act(x, act):
    if act == 'lrelu':
        return _lrelu(x, 0.01)
    return x


def _tc_linear(inputs, weights, biases, out_lrelu=False):
    nin, nout = len(inputs), len(weights)
    acts = tuple(a for _, a in inputs)

    def body(*refs):
        in_refs = refs[:nin]
        w_refs = refs[nin:nin + nin * nout]
        b_refs = refs[nin + nin * nout:nin + nin * nout + nout]
        out_refs = refs[nin + nin * nout + nout:]
        xs = [_apply_act(r[...], a) for r, a in zip(in_refs, acts)]
        for o in range(nout):
            acc = jnp.broadcast_to(b_refs[o][...], (BM, EMB))
            for i in range(nin):
                acc = acc + jnp.dot(xs[i], w_refs[o * nin + i][...],
                                    preferred_element_type=_f32)
            out_refs[o][...] = _lrelu(acc, 0.01) if out_lrelu else acc

    args = ([x for x, _ in inputs]
            + [w for per_out in weights for w in per_out]
            + [b.reshape(1, EMB) for b in biases])
    in_specs = ([_row_spec() for _ in range(nin)]
                + [_const_spec((EMB, EMB)) for _ in range(nin * nout)]
                + [_const_spec((1, EMB)) for _ in range(nout)])
    out = pl.pallas_call(
        body,
        grid=(GRID,),
        in_specs=in_specs,
        out_specs=[_row_spec() for _ in range(nout)],
        out_shape=[jax.ShapeDtypeStruct((N, EMB), _f32) for _ in range(nout)],
    )(*args)
    return out if nout > 1 else out[0]


def _tc_stats_body(x_ref, o_ref):
    @pl.when(pl.program_id(0) == 0)
    def _():
        o_ref[...] = jnp.zeros((8, EMB), _f32)
    x = x_ref[...]
    o_ref[0, :] += jnp.sum(x, axis=0)
    o_ref[1, :] += jnp.sum(x * x, axis=0)


def _tc_stats(x):
    return pl.pallas_call(
        _tc_stats_body,
        grid=(GRID,),
        in_specs=[_row_spec()],
        out_specs=pl.BlockSpec((8, EMB), lambda i: (0, 0)),
        out_shape=jax.ShapeDtypeStruct((8, EMB), _f32),
    )(x)


def _bn_from_stats(x, s_ref, g_ref, be_ref):
    mu = s_ref[0, :] * (1.0 / N)
    var = s_ref[1, :] * (1.0 / N) - mu * mu
    rstd = jax.lax.rsqrt(var + 1e-5)
    return (x - mu[None, :]) * (rstd * g_ref[0, :])[None, :] + be_ref[0, :][None, :]


def _tc_bn(x, stats, g, be, out_lrelu):
    def body(x_ref, s_ref, g_ref, b_ref, o_ref):
        y = _bn_from_stats(x_ref[...], s_ref, g_ref, b_ref)
        o_ref[...] = _lrelu(y, 0.01) if out_lrelu else y

    return pl.pallas_call(
        body,
        grid=(GRID,),
        in_specs=[_row_spec(), _const_spec((8, EMB)),
                  _const_spec((1, EMB)), _const_spec((1, EMB))],
        out_specs=_row_spec(),
        out_shape=jax.ShapeDtypeStruct((N, EMB), _f32),
    )(x, stats, g.reshape(1, EMB), be.reshape(1, EMB))


def _tc_bn_add(x1, s1, g1, be1, x2, s2, g2, be2):
    def body(x1r, s1r, g1r, b1r, x2r, s2r, g2r, b2r, o_ref):
        y1 = _lrelu(_bn_from_stats(x1r[...], s1r, g1r, b1r), 0.01)
        y2 = _lrelu(_bn_from_stats(x2r[...], s2r, g2r, b2r), 0.01)
        o_ref[...] = y1 + y2

    return pl.pallas_call(
        body,
        grid=(GRID,),
        in_specs=[_row_spec(), _const_spec((8, EMB)), _const_spec((1, EMB)),
                  _const_spec((1, EMB))] * 2,
        out_specs=_row_spec(),
        out_shape=jax.ShapeDtypeStruct((N, EMB), _f32),
    )(x1, s1, g1.reshape(1, EMB), be1.reshape(1, EMB),
      x2, s2, g2.reshape(1, EMB), be2.reshape(1, EMB))


def _tc_soc_emb(i2u_raw, ui):
    def body(a_ref, u_ref, o_ref):
        t = _lrelu(a_ref[...], 0.01)
        m = jnp.sum(t, axis=1, keepdims=True) != 0
        o_ref[...] = jnp.where(m, t, u_ref[...])

    return pl.pallas_call(
        body,
        grid=(GRID,),
        in_specs=[_row_spec(), _row_spec()],
        out_specs=_row_spec(),
        out_shape=jax.ShapeDtypeStruct((N, EMB), _f32),
    )(i2u_raw, ui)


def _tc_cheb_step1(x0, agg0, dinv2d, c1):
    def body(x0r, ar, dr, c1r, x1r, hdr):
        c = c1r[0, 0]
        d = dr[...]
        x0 = x0r[...]
        x1 = c * (x0 - ar[...] * d) - x0
        x1r[...] = x1
        hdr[...] = x1 * d

    return pl.pallas_call(
        body,
        grid=(GRID,),
        in_specs=[_row_spec(), _row_spec(), pl.BlockSpec((BM, 1), lambda i: (i, 0)),
                  pl.BlockSpec(memory_space=pltpu.SMEM)],
        out_specs=[_row_spec(), _row_spec()],
        out_shape=[jax.ShapeDtypeStruct((N, EMB), _f32)] * 2,
    )(x0, agg0, dinv2d, c1)


def _tc_cheb_step2(x1, agg1, x0, dinv2d, c1):
    def body(x1r, ar, x0r, dr, c1r, o_ref):
        c = c1r[0, 0]
        x1 = x1r[...]
        o_ref[...] = 2.0 * (c * (x1 - ar[...] * dr[...]) - x1) - x0r[...]

    return pl.pallas_call(
        body,
        grid=(GRID,),
        in_specs=[_row_spec(), _row_spec(), _row_spec(),
                  pl.BlockSpec((BM, 1), lambda i: (i, 0)),
                  pl.BlockSpec(memory_space=pltpu.SMEM)],
        out_specs=_row_spec(),
        out_shape=jax.ShapeDtypeStruct((N, EMB), _f32),
    )(x1, agg1, x0, dinv2d, c1)


def _tc_colscale(x, dinv2d):
    def body(xr, dr, o_ref):
        o_ref[...] = xr[...] * dr[...]

    return pl.pallas_call(
        body,
        grid=(GRID,),
        in_specs=[_row_spec(), pl.BlockSpec((BM, 1), lambda i: (i, 0))],
        out_specs=_row_spec(),
        out_shape=jax.ShapeDtypeStruct((N, EMB), _f32),
    )(x, dinv2d)


def _tc_mutmix(h_uP, h_uS):
    def body(pr, sr, apr, asr):
        p = pr[...]
        s = sr[...]
        hm = p * s

        def sm(x):
            ex = jnp.exp(x - jnp.max(x, axis=1, keepdims=True))
            return ex / jnp.sum(ex, axis=1, keepdims=True)

        apr[...] = hm * sm(p)
        asr[...] = hm * sm(s)

    return pl.pallas_call(
        body,
        grid=(GRID,),
        in_specs=[_row_spec(), _row_spec()],
        out_specs=[_row_spec(), _row_spec()],
        out_shape=[jax.ShapeDtypeStruct((N, EMB), _f32)] * 2,
    )(h_uP, h_uS)


def _mlp_bn_T(x_acts, p):
    k = len(x_acts)
    Ws = [p['W'][i * EMB:(i + 1) * EMB] for i in range(k)]
    T = _tc_linear(x_acts, [Ws], [p['b']])
    return T, _tc_stats(T)


def _gatv2_raw(x_act, src, dst, p):
    el, er = _tc_linear([x_act], [[p['Ws']], [p['Wd']]], [p['bs'], p['bd']])
    ee, zpart = _sc_gat_scores(el, er, p['a'], src, dst)
    z = zpart[:N] + zpart[NPAD:NPAD + N]
    zinv = 1.0 / (z + 1e-9)
    return _sc_wscatter(el, ee, zinv, src, dst, weighted=True)


def _cheb(x, src, dst, W, b, c1, dinv2d):
    zw = jnp.zeros((E,), _f32)
    zn = jnp.zeros((N,), _f32)
    X0 = x
    hd0 = _tc_colscale(X0, dinv2d)
    agg0 = _sc_wscatter(hd0, zw, zn, src, dst, weighted=False)
    X1, hd1 = _tc_cheb_step1(X0, agg0, dinv2d, c1)
    agg1 = _sc_wscatter(hd1, zw, zn, src, dst, weighted=False)
    X2 = _tc_cheb_step2(X1, agg1, X0, dinv2d, c1)
    return _tc_linear([(X0, None), (X1, None), (X2, None)],
                      [[W[0], W[1], W[2]]], [b], out_lrelu=True)


def kernel(params, laplacian_lambda_max, g_edge_index, user2item_edge_index,
           reverse_edge_index, item2user_edge_index, social_edge_index):
    lam = laplacian_lambda_max[0]
    c1 = (2.0 / lam).reshape(1, 1)

    S0 = _tc_stats(params['emb'])
    ui = _tc_bn(params['emb'], S0, params['bn_g'], params['bn_b'], out_lrelu=False)

    u2i = user2item_edge_index
    rev = reverse_edge_index
    i2u_ei = item2user_edge_index
    soc = social_edge_index

    h_raw = _gatv2_raw((ui, None), u2i[0], u2i[1], params['gat_u2i'])
    ii_raw = _gatv2_raw((h_raw, 'lrelu'), rev[0], rev[1], params['gat_ii'])
    i2u_raw = _gatv2_raw((ui, None), i2u_ei[0], i2u_ei[1], params['gat_i2u'])
    soc_emb = _tc_soc_emb(i2u_raw, ui)
    si_raw = _gatv2_raw((soc_emb, None), soc[0], soc[1], params['gat_si'])

    T_sp, S_sp = _mlp_bn_T([(ii_raw, 'lrelu'), (si_raw, 'lrelu')],
                           params['spatial_out'])
    spatial = _tc_bn(T_sp, S_sp, params['spatial_out']['g'],
                     params['spatial_out']['be'], out_lrelu=True)

    degpart = _sc_deg(soc[1])
    deg = degpart[:N] + degpart[NPAD:NPAD + N]
    dinv2d = jnp.where(deg > 0, 1.0 / jnp.sqrt(jnp.maximum(deg, 1e-9)),
                       0.0).reshape(N, 1)
    hs = _cheb(ui, soc[0], soc[1], params['cheb_W'], params['cheb_b'], c1, dinv2d)
    hs = _cheb(hs, soc[0], soc[1], params['cheb_W'], params['cheb_b'], c1, dinv2d)
    sp_raw = _gatv2_raw((hs, None), soc[0], soc[1], params['gat_spec'])

    T_P, S_P = _mlp_bn_T([(spatial, None), (ui, None)], params['mut_c'])
    h_uP = _tc_bn(T_P, S_P, params['mut_c']['g'], params['mut_c']['be'],
                  out_lrelu=True)
    T_S, S_S = _mlp_bn_T([(sp_raw, 'lrelu'), (ui, None)], params['mut_s'])
    h_uS = _tc_bn(T_S, S_S, params['mut_s']['g'], params['mut_s']['be'],
                  out_lrelu=True)

    A_P, A_S = _tc_mutmix(h_uP, h_uS)
    T_nP, S_nP = _mlp_bn_T([(A_P, None), (h_uP, None)], params['pred_p'])
    T_nS, S_nS = _mlp_bn_T([(A_S, None), (h_uS, None)], params['pred_s'])
    new_ft = _tc_bn_add(T_nP, S_nP, params['pred_p']['g'], params['pred_p']['be'],
                        T_nS, S_nS, params['pred_s']['g'], params['pred_s']['be'])

    T_r, S_r = _mlp_bn_T([(ui, None)], params['raw'])
    raw_ft = _tc_bn(T_r, S_r, params['raw']['g'], params['raw']['be'],
                    out_lrelu=True)

    ed = _sc_edge_dot(new_ft, raw_ft, g_edge_index[0], g_edge_index[1])
    return ed.reshape(E, 1)


# speedup vs baseline: 1.0950x; 1.0750x over previous
import functools

import jax
import jax.numpy as jnp
from jax import lax
from jax.experimental import pallas as pl
from jax.experimental.pallas import tpu as pltpu
from jax.experimental.pallas import tpu_sc as plsc

N_PRED = 10000
N_ITEM = 10000
N = 20000
EMB = 128
E = 320000

NC = 2
NS = 16
L = 16

NPAD = 20480
ZSTRIPE = NPAD // NS
HALF = N // NC
ACC_ROWS = 10112
ACC_STRIPE = ACC_ROWS // NS

C = 400
GRP = C // L
CS = 80
GRPS = CS // L
PT_ALL = E // (NC * NS)
PT_SC = E // NS

_mesh = plsc.VectorSubcoreMesh(core_axis_name="c", subcore_axis_name="s")
_f32 = jnp.float32
_i32 = jnp.int32


_GDN = lax.GatherDimensionNumbers(offset_dims=(), collapsed_slice_dims=(0,),
                                  start_index_map=(0,))


def _perm16(x, idx):
    return lax.gather(x, idx[:, None], _GDN, slice_sizes=(1,),
                      mode=lax.GatherScatterMode.PROMISE_IN_BOUNDS)


def _hsum_all(x):
    lane = lax.iota(_i32, L)
    for step in (8, 4, 2, 1):
        x = x + _perm16(x, lane ^ step)
    return x


def _zero_vec(ref, n):
    def body(i, _):
        ref[pl.ds(i * L, L)] = jnp.zeros((L,), _f32)
        return 0
    lax.fori_loop(0, n // L, body, 0)


def _zero_rows(ref, rows):
    def body(i, _):
        for j in range(EMB // L):
            ref[i, pl.ds(j * L, L)] = jnp.zeros((L,), _f32)
        return 0
    lax.fori_loop(0, rows, body, 0)


def _gat_scores_body(el_h, er_h, a_h, src_h, dst_h, ee_h, zp_h,
                     idxs0, idxd0, idxs1, idxd1, ee_v0, ee_v1,
                     Lb0, Rb0, Lb1, Rb1, a_v, z_sh,
                     semL0, semR0, semL1, semR1):
    c = lax.axis_index("c")
    s = lax.axis_index("s")
    wid = s * NC + c

    _zero_vec(ee_v0, CS)
    for off in range(0, ZSTRIPE, CS):
        pltpu.sync_copy(ee_v0, z_sh.at[pl.ds(s * ZSTRIPE + off, CS)])
    plsc.subcore_barrier()

    pltpu.sync_copy(a_h, a_v)

    sets = ((idxs0, idxd0, ee_v0, Lb0, Rb0, semL0, semR0),
            (idxs1, idxd1, ee_v1, Lb1, Rb1, semL1, semR1))

    def load_and_fire(t, i):
        idxs_v, idxd_v, _, Lb, Rb, sL, sR = sets[t]
        base = wid * PT_ALL + i * CS
        pltpu.sync_copy(src_h.at[pl.ds(base, CS)], idxs_v)
        pltpu.sync_copy(dst_h.at[pl.ds(base, CS)], idxd_v)
        pltpu.async_copy(el_h.at[idxs_v], Lb, sL)
        pltpu.async_copy(er_h.at[idxd_v], Rb, sR)

    def compute(t, i):
        idxs_v, idxd_v, ee_v, Lb, Rb, sL, sR = sets[t]
        base = wid * PT_ALL + i * CS
        pltpu.make_async_copy(el_h.at[idxs_v], Lb, sL).wait()
        pltpu.make_async_copy(er_h.at[idxd_v], Rb, sR).wait()

        def group(g, _):
            a_vecs = [a_v[pl.ds(j * L, L)] for j in range(EMB // L)]
            lane = jax.lax.iota(_i32, L)
            sc16 = jnp.zeros((L,), _f32)
            for k in range(L):
                row = g * L + k
                acc_l = jnp.zeros((L,), _f32)
                acc_a = jnp.zeros((L,), _f32)
                for j in range(EMB // L):
                    sv = Lb[row, pl.ds(j * L, L)] + Rb[row, pl.ds(j * L, L)]
                    acc_l = acc_l + a_vecs[j] * sv
                    acc_a = acc_a + a_vecs[j] * jnp.abs(sv)
                sk = _hsum_all(0.6 * acc_l + 0.4 * acc_a)
                sc16 = jnp.where(lane == k, sk, sc16)
            ee_v[pl.ds(g * L, L)] = jnp.exp(sc16)
            return 0

        lax.fori_loop(0, GRPS, group, 0)
        pltpu.sync_copy(ee_v, ee_h.at[pl.ds(base, CS)])
        pltpu.sync_copy(ee_v, z_sh.at[idxd_v], add=True)

    nch = PT_ALL // CS

    load_and_fire(0, 0)

    def pair(p, _):
        load_and_fire(1, 2 * p + 1)
        compute(0, 2 * p)
        load_and_fire(0, 2 * p + 2)
        compute(1, 2 * p + 1)
        return 0

    lax.fori_loop(0, nch // 2, pair, 0)
    compute(0, nch - 1)

    plsc.subcore_barrier()
    pltpu.sync_copy(z_sh.at[pl.ds(s * ZSTRIPE, ZSTRIPE)],
                    zp_h.at[pl.ds(c * NPAD + s * ZSTRIPE, ZSTRIPE)])


def _sc_gat_scores(el, er, a, src, dst):
    fn = pl.kernel(
        _gat_scores_body,
        out_type=(jax.ShapeDtypeStruct((E,), _f32),
                  jax.ShapeDtypeStruct((NC * NPAD,), _f32)),
        scratch_types=[
            pltpu.VMEM((CS,), _i32),
            pltpu.VMEM((CS,), _i32),
            pltpu.VMEM((CS,), _i32),
            pltpu.VMEM((CS,), _i32),
            pltpu.VMEM((CS,), _f32),
            pltpu.VMEM((CS,), _f32),
            pltpu.VMEM((CS, EMB), _f32),
            pltpu.VMEM((CS, EMB), _f32),
            pltpu.VMEM((CS, EMB), _f32),
            pltpu.VMEM((CS, EMB), _f32),
            pltpu.VMEM((EMB,), _f32),
            pltpu.VMEM_SHARED((NPAD,), _f32),
            pltpu.SemaphoreType.DMA,
            pltpu.SemaphoreType.DMA,
            pltpu.SemaphoreType.DMA,
            pltpu.SemaphoreType.DMA,
        ],
        mesh=_mesh,
    )
    return fn(el, er, a, src, dst)


def _deg_body(dst_h, zp_h, idxd_v, ones_v, z_sh, sem1):
    c = lax.axis_index("c")
    s = lax.axis_index("s")
    wid = s * NC + c

    _zero_vec(ones_v, C)
    for off in (0, 400, 800):
        pltpu.sync_copy(ones_v, z_sh.at[pl.ds(s * ZSTRIPE + off, 400)])
    pltpu.sync_copy(ones_v.at[pl.ds(0, 80)], z_sh.at[pl.ds(s * ZSTRIPE + 1200, 80)])
    plsc.subcore_barrier()

    def fill(i, _):
        ones_v[pl.ds(i * L, L)] = jnp.ones((L,), _f32)
        return 0
    lax.fori_loop(0, C // L, fill, 0)

    def chunk(i, _):
        base = wid * PT_ALL + i * C
        pltpu.sync_copy(dst_h.at[pl.ds(base, C)], idxd_v)
        pltpu.sync_copy(ones_v, z_sh.at[idxd_v], add=True)
        return 0

    lax.fori_loop(0, PT_ALL // C, chunk, 0)
    plsc.subcore_barrier()
    pltpu.sync_copy(z_sh.at[pl.ds(s * ZSTRIPE, ZSTRIPE)],
                    zp_h.at[pl.ds(c * NPAD + s * ZSTRIPE, ZSTRIPE)])


def _sc_deg(dst):
    fn = pl.kernel(
        _deg_body,
        out_type=jax.ShapeDtypeStruct((NC * NPAD,), _f32),
        scratch_types=[
            pltpu.VMEM((C,), _i32),
            pltpu.VMEM((C,), _f32),
            pltpu.VMEM_SHARED((NPAD,), _f32),
            pltpu.SemaphoreType.DMA,
        ],
        mesh=_mesh,
    )
    return fn(dst)


def _wscatter_body(weighted, rows_h, w_h, zinv_h, src_h, dst_h, out_h,
                   idxs0, idxd0, idxloc0, w_v0, zw_v0,
                   idxs1, idxd1, idxloc1, w_v1, zw_v1,
                   Vb0, Vb1, acc_sh, semV0, semZ0, semV1, semZ1):
    c = lax.axis_index("c")
    s = lax.axis_index("s")
    half0 = c * HALF

    _zero_rows(Vb0, CS)
    for off in range(0, ACC_STRIPE, CS):
        ln = min(CS, ACC_STRIPE - off)
        pltpu.sync_copy(Vb0.at[pl.ds(0, ln)],
                        acc_sh.at[pl.ds(s * ACC_STRIPE + off, ln)])
    plsc.subcore_barrier()

    sets = ((idxs0, idxd0, idxloc0, w_v0, zw_v0, Vb0, semV0, semZ0),
            (idxs1, idxd1, idxloc1, w_v1, zw_v1, Vb1, semV1, semZ1))

    def load_and_fire(t, i):
        idxs_v, idxd_v, _, w_v, zw_v, Vb, sV, sZ = sets[t]
        base = s * PT_SC + i * CS
        pltpu.sync_copy(src_h.at[pl.ds(base, CS)], idxs_v)
        pltpu.sync_copy(dst_h.at[pl.ds(base, CS)], idxd_v)
        if weighted:
            pltpu.sync_copy(w_h.at[pl.ds(base, CS)], w_v)
            pltpu.async_copy(zinv_h.at[idxd_v], zw_v, sZ)
        pltpu.async_copy(rows_h.at[idxs_v], Vb, sV)

    def compute(t, i):
        idxs_v, idxd_v, idxloc_v, w_v, zw_v, Vb, sV, sZ = sets[t]
        pltpu.make_async_copy(rows_h.at[idxs_v], Vb, sV).wait()
        if weighted:
            pltpu.make_async_copy(zinv_h.at[idxd_v], zw_v, sZ).wait()

        def group(g, _):
            d16 = idxd_v[pl.ds(g * L, L)]
            inhalf = (d16 >= half0) & (d16 < half0 + HALF)
            loc = jnp.where(inhalf, d16 - half0, HALF)
            idxloc_v[pl.ds(g * L, L)] = loc
            if weighted:
                wv = w_v[pl.ds(g * L, L)] * zw_v[pl.ds(g * L, L)]
                for k in range(L):
                    row = g * L + k
                    wk = wv[k]
                    for j in range(EMB // L):
                        Vb[row, pl.ds(j * L, L)] = Vb[row, pl.ds(j * L, L)] * wk
            return 0

        lax.fori_loop(0, GRPS, group, 0)
        pltpu.sync_copy(Vb, acc_sh.at[idxloc_v], add=True)

    nch = PT_SC // CS

    load_and_fire(0, 0)

    def pair(p, _):
        load_and_fire(1, 2 * p + 1)
        compute(0, 2 * p)

        @pl.when(p < nch // 2 - 1)
        def _():
            load_and_fire(0, 2 * p + 2)

        compute(1, 2 * p + 1)
        return 0

    lax.fori_loop(0, nch // 2, pair, 0)
    plsc.subcore_barrier()

    @pl.when(s < NS - 1)
    def _():
        pltpu.sync_copy(acc_sh.at[pl.ds(s * ACC_STRIPE, ACC_STRIPE)],
                        out_h.at[pl.ds(c * HALF + s * ACC_STRIPE, ACC_STRIPE)])

    @pl.when(s == NS - 1)
    def _():
        pltpu.sync_copy(acc_sh.at[pl.ds((NS - 1) * ACC_STRIPE, HALF - (NS - 1) * ACC_STRIPE)],
                        out_h.at[pl.ds(c * HALF + (NS - 1) * ACC_STRIPE, HALF - (NS - 1) * ACC_STRIPE)])


def _sc_wscatter(rows, w, zinv, src, dst, weighted):
    fn = pl.kernel(
        functools.partial(_wscatter_body, weighted),
        out_type=jax.ShapeDtypeStruct((N, EMB), _f32),
        scratch_types=(
            [pltpu.VMEM((CS,), _i32)] * 3 + [pltpu.VMEM((CS,), _f32)] * 2
            + [pltpu.VMEM((CS,), _i32)] * 3 + [pltpu.VMEM((CS,), _f32)] * 2
            + [pltpu.VMEM((CS, EMB), _f32)] * 2
            + [pltpu.VMEM_SHARED((ACC_ROWS, EMB), _f32)]
            + [pltpu.SemaphoreType.DMA] * 4
        ),
        mesh=_mesh,
    )
    return fn(rows, w, zinv, src, dst)


def _edge_dot_body(a_h, b_h, src_h, dst_h, out_h,
                   idxs0, idxd0, out_v0, Ab0, Bb0,
                   idxs1, idxd1, out_v1, Ab1, Bb1,
                   semA0, semB0, semA1, semB1):
    c = lax.axis_index("c")
    s = lax.axis_index("s")
    wid = s * NC + c

    sets = ((idxs0, idxd0, out_v0, Ab0, Bb0, semA0, semB0),
            (idxs1, idxd1, out_v1, Ab1, Bb1, semA1, semB1))

    def load_and_fire(t, i):
        idxs_v, idxd_v, _, Ab, Bb, sA, sB = sets[t]
        base = wid * PT_ALL + i * CS
        pltpu.sync_copy(src_h.at[pl.ds(base, CS)], idxs_v)
        pltpu.sync_copy(dst_h.at[pl.ds(base, CS)], idxd_v)
        pltpu.async_copy(a_h.at[idxs_v], Ab, sA)
        pltpu.async_copy(b_h.at[idxd_v], Bb, sB)

    def compute(t, i):
        idxs_v, idxd_v, out_v, Ab, Bb, sA, sB = sets[t]
        base = wid * PT_ALL + i * CS
        pltpu.make_async_copy(a_h.at[idxs_v], Ab, sA).wait()
        pltpu.make_async_copy(b_h.at[idxd_v], Bb, sB).wait()

        def group(g, _):
            lane = jax.lax.iota(_i32, L)
            dot16 = jnp.zeros((L,), _f32)
            for k in range(L):
                row = g * L + k
                acc = jnp.zeros((L,), _f32)
                for j in range(EMB // L):
                    acc = acc + Ab[row, pl.ds(j * L, L)] * Bb[row, pl.ds(j * L, L)]
                dot16 = jnp.where(lane == k, _hsum_all(acc), dot16)
            out_v[pl.ds(g * L, L)] = dot16
            return 0

        lax.fori_loop(0, GRPS, group, 0)
        pltpu.sync_copy(out_v, out_h.at[pl.ds(base, CS)])

    nch = PT_ALL // CS
    load_and_fire(0, 0)

    def pair(p, _):
        load_and_fire(1, 2 * p + 1)
        compute(0, 2 * p)
        load_and_fire(0, 2 * p + 2)
        compute(1, 2 * p + 1)
        return 0

    lax.fori_loop(0, nch // 2, pair, 0)
    compute(0, nch - 1)


def _sc_edge_dot(a, b, src, dst):
    fn = pl.kernel(
        _edge_dot_body,
        out_type=jax.ShapeDtypeStruct((E,), _f32),
        scratch_types=(
            [pltpu.VMEM((CS,), _i32)] * 2 + [pltpu.VMEM((CS,), _f32)]
            + [pltpu.VMEM((CS, EMB), _f32)] * 2
            + [pltpu.VMEM((CS,), _i32)] * 2 + [pltpu.VMEM((CS,), _f32)]
            + [pltpu.VMEM((CS, EMB), _f32)] * 2
            + [pltpu.SemaphoreType.DMA] * 4
        ),
        mesh=_mesh,
    )
    return fn(a, b, src, dst)


BM = 2000
GRID = N // BM


def _lrelu(x, s):
    return jnp.where(x >= 0, x, s * x)


def _row_spec():
    return pl.BlockSpec((BM, EMB), lambda i: (i, 0))


def _const_spec(shape):
    return pl.BlockSpec(shape, lambda i: (0, 0))


def _apply_act(x, act):
    if act == 'lrelu':
        return _lrelu(x, 0.01)
    return x


def _tc_linear(inputs, weights, biases, out_lrelu=False):
    nin, nout = len(inputs), len(weights)
    acts = tuple(a for _, a in inputs)

    def body(*refs):
        in_refs = refs[:nin]
        w_refs = refs[nin:nin + nin * nout]
        b_refs = refs[nin + nin * nout:nin + nin * nout + nout]
        out_refs = refs[nin + nin * nout + nout:]
        xs = [_apply_act(r[...], a) for r, a in zip(in_refs, acts)]
        for o in range(nout):
            acc = jnp.broadcast_to(b_refs[o][...], (BM, EMB))
            for i in range(nin):
                acc = acc + jnp.dot(xs[i], w_refs[o * nin + i][...],
                                    preferred_element_type=_f32)
            out_refs[o][...] = _lrelu(acc, 0.01) if out_lrelu else acc

    args = ([x for x, _ in inputs]
            + [w for per_out in weights for w in per_out]
            + [b.reshape(1, EMB) for b in biases])
    in_specs = ([_row_spec() for _ in range(nin)]
                + [_const_spec((EMB, EMB)) for _ in range(nin * nout)]
                + [_const_spec((1, EMB)) for _ in range(nout)])
    out = pl.pallas_call(
        body,
        grid=(GRID,),
        in_specs=in_specs,
        out_specs=[_row_spec() for _ in range(nout)],
        out_shape=[jax.ShapeDtypeStruct((N, EMB), _f32) for _ in range(nout)],
    )(*args)
    return out if nout > 1 else out[0]


def _tc_stats_body(x_ref, o_ref):
    @pl.when(pl.program_id(0) == 0)
    def _():
        o_ref[...] = jnp.zeros((8, EMB), _f32)
    x = x_ref[...]
    o_ref[0, :] += jnp.sum(x, axis=0)
    o_ref[1, :] += jnp.sum(x * x, axis=0)


def _tc_stats(x):
    return pl.pallas_call(
        _tc_stats_body,
        grid=(GRID,),
        in_specs=[_row_spec()],
        out_specs=pl.BlockSpec((8, EMB), lambda i: (0, 0)),
        out_shape=jax.ShapeDtypeStruct((8, EMB), _f32),
    )(x)


def _bn_from_stats(x, s_ref, g_ref, be_ref):
    mu = s_ref[0, :] * (1.0 / N)
    var = s_ref[1, :] * (1.0 / N) - mu * mu
    rstd = jax.lax.rsqrt(var + 1e-5)
    return (x - mu[None, :]) * (rstd * g_ref[0, :])[None, :] + be_ref[0, :][None, :]


def _tc_bn(x, stats, g, be, out_lrelu):
    def body(x_ref, s_ref, g_ref, b_ref, o_ref):
        y = _bn_from_stats(x_ref[...], s_ref, g_ref, b_ref)
        o_ref[...] = _lrelu(y, 0.01) if out_lrelu else y

    return pl.pallas_call(
        body,
        grid=(GRID,),
        in_specs=[_row_spec(), _const_spec((8, EMB)),
                  _const_spec((1, EMB)), _const_spec((1, EMB))],
        out_specs=_row_spec(),
        out_shape=jax.ShapeDtypeStruct((N, EMB), _f32),
    )(x, stats, g.reshape(1, EMB), be.reshape(1, EMB))


def _tc_bn_add(x1, s1, g1, be1, x2, s2, g2, be2):
    def body(x1r, s1r, g1r, b1r, x2r, s2r, g2r, b2r, o_ref):
        y1 = _lrelu(_bn_from_stats(x1r[...], s1r, g1r, b1r), 0.01)
        y2 = _lrelu(_bn_from_stats(x2r[...], s2r, g2r, b2r), 0.01)
        o_ref[...] = y1 + y2

    return pl.pallas_call(
        body,
        grid=(GRID,),
        in_specs=[_row_spec(), _const_spec((8, EMB)), _const_spec((1, EMB)),
                  _const_spec((1, EMB))] * 2,
        out_specs=_row_spec(),
        out_shape=jax.ShapeDtypeStruct((N, EMB), _f32),
    )(x1, s1, g1.reshape(1, EMB), be1.reshape(1, EMB),
      x2, s2, g2.reshape(1, EMB), be2.reshape(1, EMB))


def _tc_soc_emb(i2u_raw, ui):
    def body(a_ref, u_ref, o_ref):
        t = _lrelu(a_ref[...], 0.01)
        m = jnp.sum(t, axis=1, keepdims=True) != 0
        o_ref[...] = jnp.where(m, t, u_ref[...])

    return pl.pallas_call(
        body,
        grid=(GRID,),
        in_specs=[_row_spec(), _row_spec()],
        out_specs=_row_spec(),
        out_shape=jax.ShapeDtypeStruct((N, EMB), _f32),
    )(i2u_raw, ui)


def _tc_cheb_step1(x0, agg0, dinv2d, c1):
    def body(x0r, ar, dr, c1r, x1r, hdr):
        c = c1r[0, 0]
        d = dr[...]
        x0 = x0r[...]
        x1 = c * (x0 - ar[...] * d) - x0
        x1r[...] = x1
        hdr[...] = x1 * d

    return pl.pallas_call(
        body,
        grid=(GRID,),
        in_specs=[_row_spec(), _row_spec(), pl.BlockSpec((BM, 1), lambda i: (i, 0)),
                  pl.BlockSpec(memory_space=pltpu.SMEM)],
        out_specs=[_row_spec(), _row_spec()],
        out_shape=[jax.ShapeDtypeStruct((N, EMB), _f32)] * 2,
    )(x0, agg0, dinv2d, c1)


def _tc_cheb_step2(x1, agg1, x0, dinv2d, c1):
    def body(x1r, ar, x0r, dr, c1r, o_ref):
        c = c1r[0, 0]
        x1 = x1r[...]
        o_ref[...] = 2.0 * (c * (x1 - ar[...] * dr[...]) - x1) - x0r[...]

    return pl.pallas_call(
        body,
        grid=(GRID,),
        in_specs=[_row_spec(), _row_spec(), _row_spec(),
                  pl.BlockSpec((BM, 1), lambda i: (i, 0)),
                  pl.BlockSpec(memory_space=pltpu.SMEM)],
        out_specs=_row_spec(),
        out_shape=jax.ShapeDtypeStruct((N, EMB), _f32),
    )(x1, agg1, x0, dinv2d, c1)


def _tc_colscale(x, dinv2d):
    def body(xr, dr, o_ref):
        o_ref[...] = xr[...] * dr[...]

    return pl.pallas_call(
        body,
        grid=(GRID,),
        in_specs=[_row_spec(), pl.BlockSpec((BM, 1), lambda i: (i, 0))],
        out_specs=_row_spec(),
        out_shape=jax.ShapeDtypeStruct((N, EMB), _f32),
    )(x, dinv2d)


def _tc_mutmix(h_uP, h_uS):
    def body(pr, sr, apr, asr):
        p = pr[...]
        s = sr[...]
        hm = p * s

        def sm(x):
            ex = jnp.exp(x - jnp.max(x, axis=1, keepdims=True))
            return ex / jnp.sum(ex, axis=1, keepdims=True)

        apr[...] = hm * sm(p)
        asr[...] = hm * sm(s)

    return pl.pallas_call(
        body,
        grid=(GRID,),
        in_specs=[_row_spec(), _row_spec()],
        out_specs=[_row_spec(), _row_spec()],
        out_shape=[jax.ShapeDtypeStruct((N, EMB), _f32)] * 2,
    )(h_uP, h_uS)


def _mlp_bn_T(x_acts, p):
    k = len(x_acts)
    Ws = [p['W'][i * EMB:(i + 1) * EMB] for i in range(k)]
    T = _tc_linear(x_acts, [Ws], [p['b']])
    return T, _tc_stats(T)


def _gatv2_raw(x_act, src, dst, p):
    el, er = _tc_linear([x_act], [[p['Ws']], [p['Wd']]], [p['bs'], p['bd']])
    ee, zpart = _sc_gat_scores(el, er, p['a'], src, dst)
    z = zpart[:N] + zpart[NPAD:NPAD + N]
    zinv = 1.0 / (z + 1e-9)
    return _sc_wscatter(el, ee, zinv, src, dst, weighted=True)


def _cheb(x, src, dst, W, b, c1, dinv2d):
    zw = jnp.zeros((E,), _f32)
    zn = jnp.zeros((N,), _f32)
    X0 = x
    hd0 = _tc_colscale(X0, dinv2d)
    agg0 = _sc_wscatter(hd0, zw, zn, src, dst, weighted=False)
    X1, hd1 = _tc_cheb_step1(X0, agg0, dinv2d, c1)
    agg1 = _sc_wscatter(hd1, zw, zn, src, dst, weighted=False)
    X2 = _tc_cheb_step2(X1, agg1, X0, dinv2d, c1)
    return _tc_linear([(X0, None), (X1, None), (X2, None)],
                      [[W[0], W[1], W[2]]], [b], out_lrelu=True)


def kernel(params, laplacian_lambda_max, g_edge_index, user2item_edge_index,
           reverse_edge_index, item2user_edge_index, social_edge_index):
    lam = laplacian_lambda_max[0]
    c1 = (2.0 / lam).reshape(1, 1)

    S0 = _tc_stats(params['emb'])
    ui = _tc_bn(params['emb'], S0, params['bn_g'], params['bn_b'], out_lrelu=False)

    u2i = user2item_edge_index
    rev = reverse_edge_index
    i2u_ei = item2user_edge_index
    soc = social_edge_index

    h_raw = _gatv2_raw((ui, None), u2i[0], u2i[1], params['gat_u2i'])
    ii_raw = _gatv2_raw((h_raw, 'lrelu'), rev[0], rev[1], params['gat_ii'])
    i2u_raw = _gatv2_raw((ui, None), i2u_ei[0], i2u_ei[1], params['gat_i2u'])
    soc_emb = _tc_soc_emb(i2u_raw, ui)
    si_raw = _gatv2_raw((soc_emb, None), soc[0], soc[1], params['gat_si'])

    T_sp, S_sp = _mlp_bn_T([(ii_raw, 'lrelu'), (si_raw, 'lrelu')],
                           params['spatial_out'])
    spatial = _tc_bn(T_sp, S_sp, params['spatial_out']['g'],
                     params['spatial_out']['be'], out_lrelu=True)

    degpart = _sc_deg(soc[1])
    deg = degpart[:N] + degpart[NPAD:NPAD + N]
    dinv2d = jnp.where(deg > 0, 1.0 / jnp.sqrt(jnp.maximum(deg, 1e-9)),
                       0.0).reshape(N, 1)
    hs = _cheb(ui, soc[0], soc[1], params['cheb_W'], params['cheb_b'], c1, dinv2d)
    hs = _cheb(hs, soc[0], soc[1], params['cheb_W'], params['cheb_b'], c1, dinv2d)
    sp_raw = _gatv2_raw((hs, None), soc[0], soc[1], params['gat_spec'])

    T_P, S_P = _mlp_bn_T([(spatial, None), (ui, None)], params['mut_c'])
    h_uP = _tc_bn(T_P, S_P, params['mut_c']['g'], params['mut_c']['be'],
                  out_lrelu=True)
    T_S, S_S = _mlp_bn_T([(sp_raw, 'lrelu'), (ui, None)], params['mut_s'])
    h_uS = _tc_bn(T_S, S_S, params['mut_s']['g'], params['mut_s']['be'],
                  out_lrelu=True)

    A_P, A_S = _tc_mutmix(h_uP, h_uS)
    T_nP, S_nP = _mlp_bn_T([(A_P, None), (h_uP, None)], params['pred_p'])
    T_nS, S_nS = _mlp_bn_T([(A_S, None), (h_uS, None)], params['pred_s'])
    new_ft = _tc_bn_add(T_nP, S_nP, params['pred_p']['g'], params['pred_p']['be'],
                        T_nS, S_nS, params['pred_s']['g'], params['pred_s']['be'])

    T_r, S_r = _mlp_bn_T([(ui, None)], params['raw'])
    raw_ft = _tc_bn(T_r, S_r, params['raw']['g'], params['raw']['be'],
                    out_lrelu=True)

    ed = _sc_edge_dot(new_ft, raw_ft, g_edge_index[0], g_edge_index[1])
    return ed.reshape(E, 1)


# speedup vs baseline: 1.3077x; 1.1943x over previous
import functools

import jax
import jax.numpy as jnp
from jax import lax
from jax.experimental import pallas as pl
from jax.experimental.pallas import tpu as pltpu
from jax.experimental.pallas import tpu_sc as plsc

N_PRED = 10000
N_ITEM = 10000
N = 20000
EMB = 128
E = 320000

NC = 2
NS = 16
L = 16

NPAD = 20480
ZSTRIPE = NPAD // NS
HALF = N // NC
ACC_ROWS = 10112
ACC_STRIPE = ACC_ROWS // NS

C = 400
GRP = C // L
CS = 80
GRPS = CS // L
CW = 160
GRPW = CW // L
PT_ALL = E // (NC * NS)
PT_SC = E // NS

_mesh = plsc.VectorSubcoreMesh(core_axis_name="c", subcore_axis_name="s")
_f32 = jnp.float32
_i32 = jnp.int32


_GDN = lax.GatherDimensionNumbers(offset_dims=(), collapsed_slice_dims=(0,),
                                  start_index_map=(0,))


def _perm16(x, idx):
    return lax.gather(x, idx[:, None], _GDN, slice_sizes=(1,),
                      mode=lax.GatherScatterMode.PROMISE_IN_BOUNDS)


def _hsum_all(x):
    lane = lax.iota(_i32, L)
    for step in (8, 4, 2, 1):
        x = x + _perm16(x, lane ^ step)
    return x


def _zero_vec(ref, n):
    def body(i, _):
        ref[pl.ds(i * L, L)] = jnp.zeros((L,), _f32)
        return 0
    lax.fori_loop(0, n // L, body, 0)


def _zero_rows(ref, rows):
    def body(i, _):
        for j in range(EMB // L):
            ref[i, pl.ds(j * L, L)] = jnp.zeros((L,), _f32)
        return 0
    lax.fori_loop(0, rows, body, 0)


def _gat_scores_body(el_h, er_h, a_h, src_h, dst_h, ee_h, zp_h,
                     idxs0, idxd0, idxs1, idxd1, ee_v0, ee_v1,
                     Lb0, Rb0, Lb1, Rb1, a_v, z_sh,
                     semL0, semR0, semL1, semR1):
    c = lax.axis_index("c")
    s = lax.axis_index("s")
    wid = s * NC + c

    _zero_vec(ee_v0, CS)
    for off in range(0, ZSTRIPE, CS):
        pltpu.sync_copy(ee_v0, z_sh.at[pl.ds(s * ZSTRIPE + off, CS)])
    plsc.subcore_barrier()

    pltpu.sync_copy(a_h, a_v)

    sets = ((idxs0, idxd0, ee_v0, Lb0, Rb0, semL0, semR0),
            (idxs1, idxd1, ee_v1, Lb1, Rb1, semL1, semR1))

    def load_and_fire(t, i):
        idxs_v, idxd_v, _, Lb, Rb, sL, sR = sets[t]
        base = wid * PT_ALL + i * CS
        pltpu.sync_copy(src_h.at[pl.ds(base, CS)], idxs_v)
        pltpu.sync_copy(dst_h.at[pl.ds(base, CS)], idxd_v)
        pltpu.async_copy(el_h.at[idxs_v], Lb, sL)
        pltpu.async_copy(er_h.at[idxd_v], Rb, sR)

    def compute(t, i):
        idxs_v, idxd_v, ee_v, Lb, Rb, sL, sR = sets[t]
        base = wid * PT_ALL + i * CS
        pltpu.make_async_copy(el_h.at[idxs_v], Lb, sL).wait()
        pltpu.make_async_copy(er_h.at[idxd_v], Rb, sR).wait()

        def group(g, _):
            a_vecs = [a_v[pl.ds(j * L, L)] for j in range(EMB // L)]
            lane = jax.lax.iota(_i32, L)
            sc16 = jnp.zeros((L,), _f32)
            for k in range(L):
                row = g * L + k
                acc_l = jnp.zeros((L,), _f32)
                acc_a = jnp.zeros((L,), _f32)
                for j in range(EMB // L):
                    sv = Lb[row, pl.ds(j * L, L)] + Rb[row, pl.ds(j * L, L)]
                    acc_l = acc_l + a_vecs[j] * sv
                    acc_a = acc_a + a_vecs[j] * jnp.abs(sv)
                sk = _hsum_all(0.6 * acc_l + 0.4 * acc_a)
                sc16 = jnp.where(lane == k, sk, sc16)
            ee_v[pl.ds(g * L, L)] = jnp.exp(sc16)
            return 0

        lax.fori_loop(0, GRPS, group, 0)
        pltpu.sync_copy(ee_v, ee_h.at[pl.ds(base, CS)])
        pltpu.sync_copy(ee_v, z_sh.at[idxd_v], add=True)

    nch = PT_ALL // CS

    load_and_fire(0, 0)

    def pair(p, _):
        load_and_fire(1, 2 * p + 1)
        compute(0, 2 * p)
        load_and_fire(0, 2 * p + 2)
        compute(1, 2 * p + 1)
        return 0

    lax.fori_loop(0, nch // 2, pair, 0)
    compute(0, nch - 1)

    plsc.subcore_barrier()
    pltpu.sync_copy(z_sh.at[pl.ds(s * ZSTRIPE, ZSTRIPE)],
                    zp_h.at[pl.ds(c * NPAD + s * ZSTRIPE, ZSTRIPE)])


def _sc_gat_scores(el, er, a, src, dst):
    fn = pl.kernel(
        _gat_scores_body,
        out_type=(jax.ShapeDtypeStruct((E,), _f32),
                  jax.ShapeDtypeStruct((NC * NPAD,), _f32)),
        scratch_types=[
            pltpu.VMEM((CS,), _i32),
            pltpu.VMEM((CS,), _i32),
            pltpu.VMEM((CS,), _i32),
            pltpu.VMEM((CS,), _i32),
            pltpu.VMEM((CS,), _f32),
            pltpu.VMEM((CS,), _f32),
            pltpu.VMEM((CS, EMB), _f32),
            pltpu.VMEM((CS, EMB), _f32),
            pltpu.VMEM((CS, EMB), _f32),
            pltpu.VMEM((CS, EMB), _f32),
            pltpu.VMEM((EMB,), _f32),
            pltpu.VMEM_SHARED((NPAD,), _f32),
            pltpu.SemaphoreType.DMA,
            pltpu.SemaphoreType.DMA,
            pltpu.SemaphoreType.DMA,
            pltpu.SemaphoreType.DMA,
        ],
        mesh=_mesh,
    )
    return fn(el, er, a, src, dst)


def _deg_body(dst_h, zp_h, idxd_v, ones_v, z_sh, sem1):
    c = lax.axis_index("c")
    s = lax.axis_index("s")
    wid = s * NC + c

    _zero_vec(ones_v, C)
    for off in (0, 400, 800):
        pltpu.sync_copy(ones_v, z_sh.at[pl.ds(s * ZSTRIPE + off, 400)])
    pltpu.sync_copy(ones_v.at[pl.ds(0, 80)], z_sh.at[pl.ds(s * ZSTRIPE + 1200, 80)])
    plsc.subcore_barrier()

    def fill(i, _):
        ones_v[pl.ds(i * L, L)] = jnp.ones((L,), _f32)
        return 0
    lax.fori_loop(0, C // L, fill, 0)

    def chunk(i, _):
        base = wid * PT_ALL + i * C
        pltpu.sync_copy(dst_h.at[pl.ds(base, C)], idxd_v)
        pltpu.sync_copy(ones_v, z_sh.at[idxd_v], add=True)
        return 0

    lax.fori_loop(0, PT_ALL // C, chunk, 0)
    plsc.subcore_barrier()
    pltpu.sync_copy(z_sh.at[pl.ds(s * ZSTRIPE, ZSTRIPE)],
                    zp_h.at[pl.ds(c * NPAD + s * ZSTRIPE, ZSTRIPE)])


def _sc_deg(dst):
    fn = pl.kernel(
        _deg_body,
        out_type=jax.ShapeDtypeStruct((NC * NPAD,), _f32),
        scratch_types=[
            pltpu.VMEM((C,), _i32),
            pltpu.VMEM((C,), _f32),
            pltpu.VMEM_SHARED((NPAD,), _f32),
            pltpu.SemaphoreType.DMA,
        ],
        mesh=_mesh,
    )
    return fn(dst)


def _wscatter_body(weighted, rows_h, w_h, zinv_h, src_h, dst_h, out_h,
                   idxs0, idxd0, idxloc0, w_v0, zw_v0,
                   idxs1, idxd1, idxloc1, w_v1, zw_v1,
                   Vb0, Vb1, acc_sh, semV0, semZ0, semV1, semZ1):
    c = lax.axis_index("c")
    s = lax.axis_index("s")
    half0 = c * HALF

    _zero_rows(Vb0, CW)
    for off in range(0, ACC_STRIPE, CW):
        ln = min(CW, ACC_STRIPE - off)
        pltpu.sync_copy(Vb0.at[pl.ds(0, ln)],
                        acc_sh.at[pl.ds(s * ACC_STRIPE + off, ln)])
    plsc.subcore_barrier()

    sets = ((idxs0, idxd0, idxloc0, w_v0, zw_v0, Vb0, semV0, semZ0),
            (idxs1, idxd1, idxloc1, w_v1, zw_v1, Vb1, semV1, semZ1))

    def load_and_fire(t, i):
        idxs_v, idxd_v, _, w_v, zw_v, Vb, sV, sZ = sets[t]
        base = s * PT_SC + i * CW
        pltpu.sync_copy(src_h.at[pl.ds(base, CW)], idxs_v)
        pltpu.sync_copy(dst_h.at[pl.ds(base, CW)], idxd_v)
        if weighted:
            pltpu.sync_copy(w_h.at[pl.ds(base, CW)], w_v)
            pltpu.async_copy(zinv_h.at[idxd_v], zw_v, sZ)
        pltpu.async_copy(rows_h.at[idxs_v], Vb, sV)

    def compute(t, i):
        idxs_v, idxd_v, idxloc_v, w_v, zw_v, Vb, sV, sZ = sets[t]
        pltpu.make_async_copy(rows_h.at[idxs_v], Vb, sV).wait()
        if weighted:
            pltpu.make_async_copy(zinv_h.at[idxd_v], zw_v, sZ).wait()

        def group(g, _):
            d16 = idxd_v[pl.ds(g * L, L)]
            inhalf = (d16 >= half0) & (d16 < half0 + HALF)
            loc = jnp.where(inhalf, d16 - half0, HALF)
            idxloc_v[pl.ds(g * L, L)] = loc
            if weighted:
                wv = w_v[pl.ds(g * L, L)] * zw_v[pl.ds(g * L, L)]
                for k in range(L):
                    row = g * L + k
                    wk = wv[k]
                    for j in range(EMB // L):
                        Vb[row, pl.ds(j * L, L)] = Vb[row, pl.ds(j * L, L)] * wk
            return 0

        lax.fori_loop(0, GRPW, group, 0)
        pltpu.sync_copy(Vb, acc_sh.at[idxloc_v], add=True)

    nch = PT_SC // CW

    load_and_fire(0, 0)

    def pair(p, _):
        load_and_fire(1, 2 * p + 1)
        compute(0, 2 * p)
        load_and_fire(0, 2 * p + 2)
        compute(1, 2 * p + 1)
        return 0

    lax.fori_loop(0, nch // 2, pair, 0)
    compute(0, nch - 1)
    plsc.subcore_barrier()

    @pl.when(s < NS - 1)
    def _():
        pltpu.sync_copy(acc_sh.at[pl.ds(s * ACC_STRIPE, ACC_STRIPE)],
                        out_h.at[pl.ds(c * HALF + s * ACC_STRIPE, ACC_STRIPE)])

    @pl.when(s == NS - 1)
    def _():
        pltpu.sync_copy(acc_sh.at[pl.ds((NS - 1) * ACC_STRIPE, HALF - (NS - 1) * ACC_STRIPE)],
                        out_h.at[pl.ds(c * HALF + (NS - 1) * ACC_STRIPE, HALF - (NS - 1) * ACC_STRIPE)])


def _sc_wscatter(rows, w, zinv, src, dst, weighted):
    fn = pl.kernel(
        functools.partial(_wscatter_body, weighted),
        out_type=jax.ShapeDtypeStruct((N, EMB), _f32),
        scratch_types=(
            [pltpu.VMEM((CW,), _i32)] * 3 + [pltpu.VMEM((CW,), _f32)] * 2
            + [pltpu.VMEM((CW,), _i32)] * 3 + [pltpu.VMEM((CW,), _f32)] * 2
            + [pltpu.VMEM((CW, EMB), _f32)] * 2
            + [pltpu.VMEM_SHARED((ACC_ROWS, EMB), _f32)]
            + [pltpu.SemaphoreType.DMA] * 4
        ),
        mesh=_mesh,
    )
    return fn(rows, w, zinv, src, dst)


def _edge_dot_body(a_h, b_h, src_h, dst_h, out_h,
                   idxs0, idxd0, out_v0, Ab0, Bb0,
                   idxs1, idxd1, out_v1, Ab1, Bb1,
                   semA0, semB0, semA1, semB1):
    c = lax.axis_index("c")
    s = lax.axis_index("s")
    wid = s * NC + c

    sets = ((idxs0, idxd0, out_v0, Ab0, Bb0, semA0, semB0),
            (idxs1, idxd1, out_v1, Ab1, Bb1, semA1, semB1))

    def load_and_fire(t, i):
        idxs_v, idxd_v, _, Ab, Bb, sA, sB = sets[t]
        base = wid * PT_ALL + i * CS
        pltpu.sync_copy(src_h.at[pl.ds(base, CS)], idxs_v)
        pltpu.sync_copy(dst_h.at[pl.ds(base, CS)], idxd_v)
        pltpu.async_copy(a_h.at[idxs_v], Ab, sA)
        pltpu.async_copy(b_h.at[idxd_v], Bb, sB)

    def compute(t, i):
        idxs_v, idxd_v, out_v, Ab, Bb, sA, sB = sets[t]
        base = wid * PT_ALL + i * CS
        pltpu.make_async_copy(a_h.at[idxs_v], Ab, sA).wait()
        pltpu.make_async_copy(b_h.at[idxd_v], Bb, sB).wait()

        def group(g, _):
            lane = jax.lax.iota(_i32, L)
            dot16 = jnp.zeros((L,), _f32)
            for k in range(L):
                row = g * L + k
                acc = jnp.zeros((L,), _f32)
                for j in range(EMB // L):
                    acc = acc + Ab[row, pl.ds(j * L, L)] * Bb[row, pl.ds(j * L, L)]
                dot16 = jnp.where(lane == k, _hsum_all(acc), dot16)
            out_v[pl.ds(g * L, L)] = dot16
            return 0

        lax.fori_loop(0, GRPS, group, 0)
        pltpu.sync_copy(out_v, out_h.at[pl.ds(base, CS)])

    nch = PT_ALL // CS
    load_and_fire(0, 0)

    def pair(p, _):
        load_and_fire(1, 2 * p + 1)
        compute(0, 2 * p)
        load_and_fire(0, 2 * p + 2)
        compute(1, 2 * p + 1)
        return 0

    lax.fori_loop(0, nch // 2, pair, 0)
    compute(0, nch - 1)


def _sc_edge_dot(a, b, src, dst):
    fn = pl.kernel(
        _edge_dot_body,
        out_type=jax.ShapeDtypeStruct((E,), _f32),
        scratch_types=(
            [pltpu.VMEM((CS,), _i32)] * 2 + [pltpu.VMEM((CS,), _f32)]
            + [pltpu.VMEM((CS, EMB), _f32)] * 2
            + [pltpu.VMEM((CS,), _i32)] * 2 + [pltpu.VMEM((CS,), _f32)]
            + [pltpu.VMEM((CS, EMB), _f32)] * 2
            + [pltpu.SemaphoreType.DMA] * 4
        ),
        mesh=_mesh,
    )
    return fn(a, b, src, dst)


BM = 2000
GRID = N // BM


def _lrelu(x, s):
    return jnp.where(x >= 0, x, s * x)


def _row_spec():
    return pl.BlockSpec((BM, EMB), lambda i: (i, 0))


def _const_spec(shape):
    return pl.BlockSpec(shape, lambda i: (0, 0))


def _apply_act(x, act):
    if act == 'lrelu':
        return _lrelu(x, 0.01)
    return x


def _tc_linear(inputs, weights, biases, out_lrelu=False):
    nin, nout = len(inputs), len(weights)
    acts = tuple(a for _, a in inputs)

    def body(*refs):
        in_refs = refs[:nin]
        w_refs = refs[nin:nin + nin * nout]
        b_refs = refs[nin + nin * nout:nin + nin * nout + nout]
        out_refs = refs[nin + nin * nout + nout:]
        xs = [_apply_act(r[...], a) for r, a in zip(in_refs, acts)]
        for o in range(nout):
            acc = jnp.broadcast_to(b_refs[o][...], (BM, EMB))
            for i in range(nin):
                acc = acc + jnp.dot(xs[i], w_refs[o * nin + i][...],
                                    preferred_element_type=_f32)
            out_refs[o][...] = _lrelu(acc, 0.01) if out_lrelu else acc

    args = ([x for x, _ in inputs]
            + [w for per_out in weights for w in per_out]
            + [b.reshape(1, EMB) for b in biases])
    in_specs = ([_row_spec() for _ in range(nin)]
                + [_const_spec((EMB, EMB)) for _ in range(nin * nout)]
                + [_const_spec((1, EMB)) for _ in range(nout)])
    out = pl.pallas_call(
        body,
        grid=(GRID,),
        in_specs=in_specs,
        out_specs=[_row_spec() for _ in range(nout)],
        out_shape=[jax.ShapeDtypeStruct((N, EMB), _f32) for _ in range(nout)],
    )(*args)
    return out if nout > 1 else out[0]


def _tc_stats_body(x_ref, o_ref):
    @pl.when(pl.program_id(0) == 0)
    def _():
        o_ref[...] = jnp.zeros((8, EMB), _f32)
    x = x_ref[...]
    o_ref[0, :] += jnp.sum(x, axis=0)
    o_ref[1, :] += jnp.sum(x * x, axis=0)


def _tc_stats(x):
    return pl.pallas_call(
        _tc_stats_body,
        grid=(GRID,),
        in_specs=[_row_spec()],
        out_specs=pl.BlockSpec((8, EMB), lambda i: (0, 0)),
        out_shape=jax.ShapeDtypeStruct((8, EMB), _f32),
    )(x)


def _bn_from_stats(x, s_ref, g_ref, be_ref):
    mu = s_ref[0, :] * (1.0 / N)
    var = s_ref[1, :] * (1.0 / N) - mu * mu
    rstd = jax.lax.rsqrt(var + 1e-5)
    return (x - mu[None, :]) * (rstd * g_ref[0, :])[None, :] + be_ref[0, :][None, :]


def _tc_bn(x, stats, g, be, out_lrelu):
    def body(x_ref, s_ref, g_ref, b_ref, o_ref):
        y = _bn_from_stats(x_ref[...], s_ref, g_ref, b_ref)
        o_ref[...] = _lrelu(y, 0.01) if out_lrelu else y

    return pl.pallas_call(
        body,
        grid=(GRID,),
        in_specs=[_row_spec(), _const_spec((8, EMB)),
                  _const_spec((1, EMB)), _const_spec((1, EMB))],
        out_specs=_row_spec(),
        out_shape=jax.ShapeDtypeStruct((N, EMB), _f32),
    )(x, stats, g.reshape(1, EMB), be.reshape(1, EMB))


def _tc_bn_add(x1, s1, g1, be1, x2, s2, g2, be2):
    def body(x1r, s1r, g1r, b1r, x2r, s2r, g2r, b2r, o_ref):
        y1 = _lrelu(_bn_from_stats(x1r[...], s1r, g1r, b1r), 0.01)
        y2 = _lrelu(_bn_from_stats(x2r[...], s2r, g2r, b2r), 0.01)
        o_ref[...] = y1 + y2

    return pl.pallas_call(
        body,
        grid=(GRID,),
        in_specs=[_row_spec(), _const_spec((8, EMB)), _const_spec((1, EMB)),
                  _const_spec((1, EMB))] * 2,
        out_specs=_row_spec(),
        out_shape=jax.ShapeDtypeStruct((N, EMB), _f32),
    )(x1, s1, g1.reshape(1, EMB), be1.reshape(1, EMB),
      x2, s2, g2.reshape(1, EMB), be2.reshape(1, EMB))


def _tc_soc_emb(i2u_raw, ui):
    def body(a_ref, u_ref, o_ref):
        t = _lrelu(a_ref[...], 0.01)
        m = jnp.sum(t, axis=1, keepdims=True) != 0
        o_ref[...] = jnp.where(m, t, u_ref[...])

    return pl.pallas_call(
        body,
        grid=(GRID,),
        in_specs=[_row_spec(), _row_spec()],
        out_specs=_row_spec(),
        out_shape=jax.ShapeDtypeStruct((N, EMB), _f32),
    )(i2u_raw, ui)


def _tc_cheb_step1(x0, agg0, dinv2d, c1):
    def body(x0r, ar, dr, c1r, x1r, hdr):
        c = c1r[0, 0]
        d = dr[...]
        x0 = x0r[...]
        x1 = c * (x0 - ar[...] * d) - x0
        x1r[...] = x1
        hdr[...] = x1 * d

    return pl.pallas_call(
        body,
        grid=(GRID,),
        in_specs=[_row_spec(), _row_spec(), pl.BlockSpec((BM, 1), lambda i: (i, 0)),
                  pl.BlockSpec(memory_space=pltpu.SMEM)],
        out_specs=[_row_spec(), _row_spec()],
        out_shape=[jax.ShapeDtypeStruct((N, EMB), _f32)] * 2,
    )(x0, agg0, dinv2d, c1)


def _tc_cheb_step2(x1, agg1, x0, dinv2d, c1):
    def body(x1r, ar, x0r, dr, c1r, o_ref):
        c = c1r[0, 0]
        x1 = x1r[...]
        o_ref[...] = 2.0 * (c * (x1 - ar[...] * dr[...]) - x1) - x0r[...]

    return pl.pallas_call(
        body,
        grid=(GRID,),
        in_specs=[_row_spec(), _row_spec(), _row_spec(),
                  pl.BlockSpec((BM, 1), lambda i: (i, 0)),
                  pl.BlockSpec(memory_space=pltpu.SMEM)],
        out_specs=_row_spec(),
        out_shape=jax.ShapeDtypeStruct((N, EMB), _f32),
    )(x1, agg1, x0, dinv2d, c1)


def _tc_colscale(x, dinv2d):
    def body(xr, dr, o_ref):
        o_ref[...] = xr[...] * dr[...]

    return pl.pallas_call(
        body,
        grid=(GRID,),
        in_specs=[_row_spec(), pl.BlockSpec((BM, 1), lambda i: (i, 0))],
        out_specs=_row_spec(),
        out_shape=jax.ShapeDtypeStruct((N, EMB), _f32),
    )(x, dinv2d)


def _tc_mutmix(h_uP, h_uS):
    def body(pr, sr, apr, asr):
        p = pr[...]
        s = sr[...]
        hm = p * s

        def sm(x):
            ex = jnp.exp(x - jnp.max(x, axis=1, keepdims=True))
            return ex / jnp.sum(ex, axis=1, keepdims=True)

        apr[...] = hm * sm(p)
        asr[...] = hm * sm(s)

    return pl.pallas_call(
        body,
        grid=(GRID,),
        in_specs=[_row_spec(), _row_spec()],
        out_specs=[_row_spec(), _row_spec()],
        out_shape=[jax.ShapeDtypeStruct((N, EMB), _f32)] * 2,
    )(h_uP, h_uS)


def _mlp_bn_T(x_acts, p):
    k = len(x_acts)
    Ws = [p['W'][i * EMB:(i + 1) * EMB] for i in range(k)]
    T = _tc_linear(x_acts, [Ws], [p['b']])
    return T, _tc_stats(T)


def _gatv2_raw(x_act, src, dst, p):
    el, er = _tc_linear([x_act], [[p['Ws']], [p['Wd']]], [p['bs'], p['bd']])
    ee, zpart = _sc_gat_scores(el, er, p['a'], src, dst)
    z = zpart[:N] + zpart[NPAD:NPAD + N]
    zinv = 1.0 / (z + 1e-9)
    return _sc_wscatter(el, ee, zinv, src, dst, weighted=True)


def _cheb(x, src, dst, W, b, c1, dinv2d):
    zw = jnp.zeros((E,), _f32)
    zn = jnp.zeros((N,), _f32)
    X0 = x
    hd0 = _tc_colscale(X0, dinv2d)
    agg0 = _sc_wscatter(hd0, zw, zn, src, dst, weighted=False)
    X1, hd1 = _tc_cheb_step1(X0, agg0, dinv2d, c1)
    agg1 = _sc_wscatter(hd1, zw, zn, src, dst, weighted=False)
    X2 = _tc_cheb_step2(X1, agg1, X0, dinv2d, c1)
    return _tc_linear([(X0, None), (X1, None), (X2, None)],
                      [[W[0], W[1], W[2]]], [b], out_lrelu=True)


def kernel(params, laplacian_lambda_max, g_edge_index, user2item_edge_index,
           reverse_edge_index, item2user_edge_index, social_edge_index):
    lam = laplacian_lambda_max[0]
    c1 = (2.0 / lam).reshape(1, 1)

    S0 = _tc_stats(params['emb'])
    ui = _tc_bn(params['emb'], S0, params['bn_g'], params['bn_b'], out_lrelu=False)

    u2i = user2item_edge_index
    rev = reverse_edge_index
    i2u_ei = item2user_edge_index
    soc = social_edge_index

    h_raw = _gatv2_raw((ui, None), u2i[0], u2i[1], params['gat_u2i'])
    ii_raw = _gatv2_raw((h_raw, 'lrelu'), rev[0], rev[1], params['gat_ii'])
    i2u_raw = _gatv2_raw((ui, None), i2u_ei[0], i2u_ei[1], params['gat_i2u'])
    soc_emb = _tc_soc_emb(i2u_raw, ui)
    si_raw = _gatv2_raw((soc_emb, None), soc[0], soc[1], params['gat_si'])

    T_sp, S_sp = _mlp_bn_T([(ii_raw, 'lrelu'), (si_raw, 'lrelu')],
                           params['spatial_out'])
    spatial = _tc_bn(T_sp, S_sp, params['spatial_out']['g'],
                     params['spatial_out']['be'], out_lrelu=True)

    degpart = _sc_deg(soc[1])
    deg = degpart[:N] + degpart[NPAD:NPAD + N]
    dinv2d = jnp.where(deg > 0, 1.0 / jnp.sqrt(jnp.maximum(deg, 1e-9)),
                       0.0).reshape(N, 1)
    hs = _cheb(ui, soc[0], soc[1], params['cheb_W'], params['cheb_b'], c1, dinv2d)
    hs = _cheb(hs, soc[0], soc[1], params['cheb_W'], params['cheb_b'], c1, dinv2d)
    sp_raw = _gatv2_raw((hs, None), soc[0], soc[1], params['gat_spec'])

    T_P, S_P = _mlp_bn_T([(spatial, None), (ui, None)], params['mut_c'])
    h_uP = _tc_bn(T_P, S_P, params['mut_c']['g'], params['mut_c']['be'],
                  out_lrelu=True)
    T_S, S_S = _mlp_bn_T([(sp_raw, 'lrelu'), (ui, None)], params['mut_s'])
    h_uS = _tc_bn(T_S, S_S, params['mut_s']['g'], params['mut_s']['be'],
                  out_lrelu=True)

    A_P, A_S = _tc_mutmix(h_uP, h_uS)
    T_nP, S_nP = _mlp_bn_T([(A_P, None), (h_uP, None)], params['pred_p'])
    T_nS, S_nS = _mlp_bn_T([(A_S, None), (h_uS, None)], params['pred_s'])
    new_ft = _tc_bn_add(T_nP, S_nP, params['pred_p']['g'], params['pred_p']['be'],
                        T_nS, S_nS, params['pred_s']['g'], params['pred_s']['be'])

    T_r, S_r = _mlp_bn_T([(ui, None)], params['raw'])
    raw_ft = _tc_bn(T_r, S_r, params['raw']['g'], params['raw']['be'],
                    out_lrelu=True)

    ed = _sc_edge_dot(new_ft, raw_ft, g_edge_index[0], g_edge_index[1])
    return ed.reshape(E, 1)
